# baseline (speedup 1.0000x reference)
import dataclasses
import functools

import jax
import jax.numpy as jnp
from jax import lax
from jax.experimental import pallas as pl
from jax.experimental.pallas import tpu as pltpu
from jax.experimental.pallas import tpu_sc as plsc

_NV, _NR = 10000, 500
_EV = 320000
_NVP = 10240
_NRP = 512
_NHALF = 5120
_ACC_VL = 6144
_ACC_R = 2048
_C = 128
_EPS = 1e-16

_SC_PARAMS = pltpu.CompilerParams()
if "needs_layout_passes" in pltpu.CompilerParams.__dataclass_fields__:
    _SC_PARAMS = dataclasses.replace(_SC_PARAMS, needs_layout_passes=False)


def _edge_pass(heads, n_dst_pad, e_pad, split_dst, n_half=0):
    nworkers = 16 if split_dst else 32
    ept = e_pad // nworkers
    nchunks = ept // _C
    rows_sub = n_dst_pad // 16
    ncopies = rows_sub // _C
    nd2r = (n_dst_pad * 2) // 128
    mesh = plsc.VectorSubcoreMesh(core_axis_name="c", subcore_axis_name="s",
                                  num_cores=2, num_subcores=16)

    def body(xl_hbm, xr_hbm, ed_hbm, w_hbm, num_hbm, den_hbm,
             ed0, ed1, dst0, dst1, w_v, xl0, xl1, xr_v, msg0, den_t,
             idx_r, acc_sh, accd_sh,
             semi0, semi1, semg0, semg1, semx):
        cid = lax.axis_index("c")
        sid = lax.axis_index("s")
        wid = sid if split_dst else cid * 16 + sid
        lane = lax.iota(jnp.int32, 16)
        lane01 = jnp.minimum(lane, 1)
        zeros = jnp.zeros((16,), jnp.float32)
        two = jnp.full((16,), 2, jnp.int32)
        three = jnp.full((16,), 3, jnp.int32)
        shuf = [lax.bitwise_xor(lane, s) for s in (1, 2, 4, 8)]
        ed = [ed0, ed1]
        dst2 = [dst0, dst1]
        xl = [xl0, xl1]
        semi = [semi0, semi1]
        semg = [semg0, semg1]

        def _bsum(v):
            for ix in shuf:
                v = v + v.at[ix].get(mode='promise_in_bounds')
            return v

        @pl.loop(0, _C)
        def _z(i):
            for j in range(8):
                msg0[i, pl.ds(j * 16, 16)] = zeros

        @pl.loop(0, ncopies)
        def _za(k):
            r0 = sid * rows_sub + k * _C
            pltpu.sync_copy(msg0, acc_sh.at[pl.ds(r0, _C)])

        @pl.when(sid == 0)
        def _zd():
            off = 0
            while off < nd2r:
                sz = min(_C, nd2r - off)
                pltpu.sync_copy(msg0.at[pl.ds(0, sz)], accd_sh.at[pl.ds(off, sz)])
                off += sz

        @pl.loop(0, nd2r)
        def _zt(i):
            for j in range(8):
                den_t[i, pl.ds(j * 16, 16)] = zeros

        @pl.loop(0, nd2r // 16)
        def _zi(k):
            idx_r[pl.ds(k * 16, 16)] = lane + k * 16

        pltpu.sync_copy(w_hbm, w_v)
        plsc.subcore_barrier()

        att = [w_v[0, pl.ds(j * 16, 16)] for j in range(8)]
        we0 = [w_v[1, pl.ds(j * 16, 16)] for j in range(8)]
        we1 = [w_v[2, pl.ds(j * 16, 16)] for j in range(8)]

        def fetch_ed(i, b):
            blk = jnp.minimum(wid * nchunks + i, (wid + 1) * nchunks - 1)
            pltpu.async_copy(ed_hbm.at[blk], ed[b], semi[b])

        def wait_ed(b):
            pltpu.make_async_copy(ed_hbm.at[0], ed[b], semi[b]).wait()

        def fire_xl(b):
            pltpu.async_copy(xl_hbm.at[ed[b].at[0]], xl[b], semg[b])

        def wait_xl(b):
            pltpu.make_async_copy(xl_hbm.at[ed[b].at[0]], xl[b], semg[b]).wait()

        def fire_xr(b):
            pltpu.async_copy(xr_hbm.at[ed[b].at[1]], xr_v, semx)

        def wait_xr(b):
            pltpu.make_async_copy(xr_hbm.at[ed[b].at[1]], xr_v, semx).wait()

        def compute(b):
            if split_dst:
                @pl.loop(0, _C // 16)
                def _loc(g):
                    d = ed[b][1, pl.ds(g * 16, 16)] - cid * n_half
                    ok = (d >= 0) & (d < n_half)
                    dst2[b][pl.ds(g * 16, 16)] = jnp.where(ok, d, n_half)
            else:
                @pl.loop(0, _C // 16)
                def _loc(g):
                    dst2[b][pl.ds(g * 16, 16)] = ed[b][1, pl.ds(g * 16, 16)]

            @pl.loop(0, _C)
            def _edge(e):
                ei = jnp.full((16,), e, jnp.int32)
                a0 = plsc.bitcast(plsc.load_gather(ed[b], [two, ei]), jnp.float32)
                a1 = plsc.bitcast(plsc.load_gather(ed[b], [three, ei]), jnp.float32)
                dstv = plsc.load_gather(dst2[b], [ei])
                xls = []
                s0 = zeros
                s1 = zeros
                for j in range(8):
                    xlj = xl[b][e, pl.ds(j * 16, 16)]
                    xrj = xr_v[e, pl.ds(j * 16, 16)]
                    m = xlj + xrj + a0 * we0[j] + a1 * we1[j]
                    m = jnp.maximum(m, 0.2 * m)
                    if heads == 2 and j >= 4:
                        s1 = s1 + m * att[j]
                    else:
                        s0 = s0 + m * att[j]
                    xls.append(xlj)
                if heads == 2:
                    e0 = jnp.exp(_bsum(s0))
                    e1 = jnp.exp(_bsum(s1))
                    for j in range(8):
                        msg0[e, pl.ds(j * 16, 16)] = xls[j] * (e0 if j < 4 else e1)
                    den = jnp.where(lane == 0, e0, jnp.where(lane == 1, e1, zeros))
                else:
                    e0 = jnp.exp(_bsum(s0))
                    for j in range(8):
                        msg0[e, pl.ds(j * 16, 16)] = xls[j] * e0
                    den = jnp.where(lane == 0, e0, zeros)
                flat = dstv * 2 + lane01
                plsc.addupdate_scatter(
                    den_t,
                    [lax.shift_right_logical(flat, 7),
                     lax.bitwise_and(flat, 127)],
                    den, mask=lane < heads)

        fetch_ed(0, 0)
        fetch_ed(1, 1)

        @pl.loop(0, nchunks // 2)
        def _pair(tt):
            for b in (0, 1):
                i = tt * 2 + b
                wait_ed(b)
                g1 = pltpu.async_copy(xl_hbm.at[ed[b].at[0]], xl[b], semg[b])
                g2 = pltpu.async_copy(xr_hbm.at[ed[b].at[1]], xr_v, semx)
                g1.wait()
                g2.wait()
                compute(b)
                pltpu.sync_copy(msg0, acc_sh.at[dst2[b]], add=True)
                fetch_ed(i + 2, b)

        wait_ed(0)
        wait_ed(1)

        pltpu.sync_copy(den_t, accd_sh.at[idx_r], add=True)
        plsc.subcore_barrier()

        @pl.loop(0, ncopies)
        def _wb(k):
            r0 = sid * rows_sub + k * _C
            pltpu.sync_copy(acc_sh.at[pl.ds(r0, _C)], num_hbm.at[cid, pl.ds(r0, _C)])

        @pl.when(sid == 0)
        def _wbd():
            pltpu.sync_copy(accd_sh, den_hbm.at[cid])

    return pl.kernel(
        body,
        out_type=(jax.ShapeDtypeStruct((2, n_dst_pad, 128), jnp.float32),
                  jax.ShapeDtypeStruct((2, nd2r, 128), jnp.float32)),
        mesh=mesh,
        compiler_params=_SC_PARAMS,
        scratch_types=[
            pltpu.VMEM((4, 128), jnp.int32),
            pltpu.VMEM((4, 128), jnp.int32),
            pltpu.VMEM((_C,), jnp.int32),
            pltpu.VMEM((_C,), jnp.int32),
            pltpu.VMEM((3, 128), jnp.float32),
            pltpu.VMEM((_C, 128), jnp.float32),
            pltpu.VMEM((_C, 128), jnp.float32),
            pltpu.VMEM((_C, 128), jnp.float32),
            pltpu.VMEM((_C, 128), jnp.float32),
            pltpu.VMEM((nd2r, 128), jnp.float32),
            pltpu.VMEM((nd2r,), jnp.int32),
            pltpu.VMEM_SHARED((n_dst_pad, 128), jnp.float32),
            pltpu.VMEM_SHARED((nd2r, 128), jnp.float32),
            pltpu.SemaphoreType.DMA,
            pltpu.SemaphoreType.DMA,
            pltpu.SemaphoreType.DMA,
            pltpu.SemaphoreType.DMA,
            pltpu.SemaphoreType.DMA,
        ],
    )


def _mm(x, w, b, bm):
    m, k = x.shape
    n = w.shape[1]
    nout = n // 128

    def body(x_ref, w_ref, b_ref, *outs):
        y = jnp.dot(x_ref[...], w_ref[...], preferred_element_type=jnp.float32)
        y = y + b_ref[...]
        for i, o in enumerate(outs):
            o[...] = y[:, i * 128:(i + 1) * 128]

    return pl.pallas_call(
        body,
        grid=(m // bm,),
        in_specs=[
            pl.BlockSpec((bm, k), lambda i: (i, 0)),
            pl.BlockSpec((k, n), lambda i: (0, 0)),
            pl.BlockSpec((1, n), lambda i: (0, 0)),
        ],
        out_specs=[pl.BlockSpec((bm, 128), lambda i: (i, 0)) for _ in range(nout)],
        out_shape=[jax.ShapeDtypeStruct((m, 128), jnp.float32) for _ in range(nout)],
    )(x, w, b.reshape(1, n))


def _total_sum(x2d):
    def body(x_ref, o_ref):
        o_ref[...] = jnp.sum(x_ref[...])[None, None]

    return pl.pallas_call(
        body, out_shape=jax.ShapeDtypeStruct((1, 1), jnp.float32))(x2d)


def _combine1_veh(numv, denv, numi, deni, xl, xr, w, s, bias_v, bias_i):
    bm = 1024

    def body(nv_ref, dv_ref, ni_ref, di_ref, xl_ref, xr_ref, w_ref, s_ref,
             bv_ref, bi_ref, o_ref):
        xl_ = xl_ref[...]
        w_ = w_ref[...]
        xe = (s_ref[0, 0] / _EV) * w_[1:2, :] + (s_ref[0, 1] / _EV) * w_[2:3, :]
        m = xl_ + xr_ref[...] + xe
        m = jnp.maximum(m, 0.2 * m)
        am = m * w_[0:1, :]
        e0 = jnp.exp(jnp.sum(am[:, :64], axis=1, keepdims=True))
        e1 = jnp.exp(jnp.sum(am[:, 64:], axis=1, keepdims=True))
        av = nv_ref[...]
        dv = dv_ref[...]
        num = av + jnp.concatenate([e0 * xl_[:, :64], e1 * xl_[:, 64:]], axis=1)
        ov = jnp.concatenate(
            [num[:, :64] / (dv[:, 0:1] + e0 + _EPS),
             num[:, 64:] / (dv[:, 1:2] + e1 + _EPS)], axis=1) + bv_ref[...]
        ai = ni_ref[...]
        di = di_ref[...]
        oi = jnp.concatenate(
            [ai[:, :64] / (di[:, 0:1] + _EPS),
             ai[:, 64:] / (di[:, 1:2] + _EPS)], axis=1) + bi_ref[...]
        v = ov + oi
        o_ref[...] = jnp.where(v > 0, v, jnp.exp(v) - 1.0)

    return pl.pallas_call(
        body,
        grid=(_NVP // bm,),
        in_specs=[
            pl.BlockSpec((bm, 128), lambda i: (i, 0)),
            pl.BlockSpec((bm, 2), lambda i: (i, 0)),
            pl.BlockSpec((bm, 128), lambda i: (i, 0)),
            pl.BlockSpec((bm, 2), lambda i: (i, 0)),
            pl.BlockSpec((bm, 128), lambda i: (i, 0)),
            pl.BlockSpec((bm, 128), lambda i: (i, 0)),
            pl.BlockSpec((3, 128), lambda i: (0, 0)),
            pl.BlockSpec(memory_space=pltpu.SMEM),
            pl.BlockSpec((1, 128), lambda i: (0, 0)),
            pl.BlockSpec((1, 128), lambda i: (0, 0)),
        ],
        out_specs=pl.BlockSpec((bm, 128), lambda i: (i, 0)),
        out_shape=jax.ShapeDtypeStruct((_NVP, 128), jnp.float32),
    )(numv, denv, numi, deni, xl, xr, w, s, bias_v, bias_i)


def _combine1_rsu(num, den, bias):
    def body(n_ref, d_ref, b_ref, o_ref):
        a = n_ref[0] + n_ref[1]
        d = d_ref[0] + d_ref[1]
        o = jnp.concatenate(
            [a[:, :64] / (d[:, 0:1] + _EPS),
             a[:, 64:] / (d[:, 1:2] + _EPS)], axis=1) + b_ref[...]
        o_ref[...] = jnp.where(o > 0, o, jnp.exp(o) - 1.0)

    return pl.pallas_call(
        body,
        grid=(1,),
        in_specs=[
            pl.BlockSpec((2, _NRP, 128), lambda i: (0, 0, 0)),
            pl.BlockSpec((2, _NRP, 2), lambda i: (0, 0, 0)),
            pl.BlockSpec((1, 128), lambda i: (0, 0)),
        ],
        out_specs=pl.BlockSpec((_NRP, 128), lambda i: (0, 0)),
        out_shape=jax.ShapeDtypeStruct((_NRP, 128), jnp.float32),
    )(num, den, bias)


def _combine2_veh(numv, denv, numi, deni, xl, xr, w, s, bias_v, bias_i, g, b):
    bm = 1024

    def body(nv_ref, dv_ref, ni_ref, di_ref, xl_ref, xr_ref, w_ref, s_ref,
             bv_ref, bi_ref, g_ref, be_ref, o_ref):
        xl_ = xl_ref[...]
        w_ = w_ref[...]
        xe = (s_ref[0, 0] / _EV) * w_[1:2, :] + (s_ref[0, 1] / _EV) * w_[2:3, :]
        m = xl_ + xr_ref[...] + xe
        m = jnp.maximum(m, 0.2 * m)
        e = jnp.exp(jnp.sum(m * w_[0:1, :], axis=1, keepdims=True))
        av = nv_ref[...]
        dv = dv_ref[...]
        ov = (av + e * xl_) / (dv[:, 0:1] + e + _EPS) + bv_ref[...]
        ai = ni_ref[...]
        di = di_ref[...]
        oi = ai / (di[:, 0:1] + _EPS) + bi_ref[...]
        v = ov + oi
        mu = jnp.mean(v, axis=1, keepdims=True)
        cv = v - mu
        var = jnp.mean(cv * cv, axis=1, keepdims=True)
        o_ref[...] = cv * lax.rsqrt(var + 1e-5) * g_ref[...] + be_ref[...]

    return pl.pallas_call(
        body,
        grid=(_NVP // bm,),
        in_specs=[
            pl.BlockSpec((bm, 128), lambda i: (i, 0)),
            pl.BlockSpec((bm, 2), lambda i: (i, 0)),
            pl.BlockSpec((bm, 128), lambda i: (i, 0)),
            pl.BlockSpec((bm, 2), lambda i: (i, 0)),
            pl.BlockSpec((bm, 128), lambda i: (i, 0)),
            pl.BlockSpec((bm, 128), lambda i: (i, 0)),
            pl.BlockSpec((3, 128), lambda i: (0, 0)),
            pl.BlockSpec(memory_space=pltpu.SMEM),
            pl.BlockSpec((1, 128), lambda i: (0, 0)),
            pl.BlockSpec((1, 128), lambda i: (0, 0)),
            pl.BlockSpec((1, 128), lambda i: (0, 0)),
            pl.BlockSpec((1, 128), lambda i: (0, 0)),
        ],
        out_specs=pl.BlockSpec((bm, 128), lambda i: (i, 0)),
        out_shape=jax.ShapeDtypeStruct((_NVP, 128), jnp.float32),
    )(numv, denv, numi, deni, xl, xr, w, s, bias_v, bias_i, g, b)


def _combine2_rsu(num, den, bias, g, b):
    def body(n_ref, d_ref, b_ref, g_ref, be_ref, o_ref):
        a = n_ref[0] + n_ref[1]
        d = d_ref[0] + d_ref[1]
        v = a / (d[:, 0:1] + _EPS) + b_ref[...]
        mu = jnp.mean(v, axis=1, keepdims=True)
        cv = v - mu
        var = jnp.mean(cv * cv, axis=1, keepdims=True)
        o_ref[...] = cv * lax.rsqrt(var + 1e-5) * g_ref[...] + be_ref[...]

    return pl.pallas_call(
        body,
        grid=(1,),
        in_specs=[
            pl.BlockSpec((2, _NRP, 128), lambda i: (0, 0, 0)),
            pl.BlockSpec((2, _NRP, 2), lambda i: (0, 0, 0)),
            pl.BlockSpec((1, 128), lambda i: (0, 0)),
            pl.BlockSpec((1, 128), lambda i: (0, 0)),
            pl.BlockSpec((1, 128), lambda i: (0, 0)),
        ],
        out_specs=pl.BlockSpec((_NRP, 128), lambda i: (0, 0)),
        out_shape=jax.ShapeDtypeStruct((_NRP, 128), jnp.float32),
    )(num, den, bias, g, b)


def _pad_edges(src, dst, ea, n_dst):
    e = src.shape[0]
    ep = -(-e // 8192) * 8192
    pad = ep - e
    src_p = jnp.concatenate([src.astype(jnp.int32), jnp.zeros((pad,), jnp.int32)])
    dst_p = jnp.concatenate([dst.astype(jnp.int32),
                             jnp.full((pad,), n_dst, jnp.int32)])
    ea0 = jnp.concatenate([ea[:, 0], jnp.zeros((pad,), jnp.float32)])
    ea1 = jnp.concatenate([ea[:, 1], jnp.zeros((pad,), jnp.float32)])
    edata = jnp.stack([src_p.reshape(-1, 128),
                       dst_p.reshape(-1, 128),
                       lax.bitcast_convert_type(ea0, jnp.int32).reshape(-1, 128),
                       lax.bitcast_convert_type(ea1, jnp.int32).reshape(-1, 128)],
                      axis=1)
    return edata, ep


def _wmat(cp):
    return jnp.stack([cp['att'].reshape(-1), cp['We'][0], cp['We'][1]])


def kernel(x_vehicle, x_rsu, edge_index_v2v, v2i_src, v2i_dst, i2v_src, i2v_dst,
           edge_attr_v2v, edge_attr_v2i, edge_attr_i2v, params):
    p = params
    xv = jnp.pad(x_vehicle, ((0, _NVP - _NV), (0, 0)))
    xu = jnp.pad(x_rsu, ((0, _NRP - _NR), (0, 0)))
    sv, dv = edge_index_v2v[0], edge_index_v2v[1]

    edv, epv = _pad_edges(sv, dv, edge_attr_v2v, _NV)
    edi, epi = _pad_edges(i2v_src, i2v_dst, edge_attr_i2v, _NV)
    edb, epb = _pad_edges(v2i_src, v2i_dst, edge_attr_v2i, _NR)

    s0 = _total_sum(edge_attr_v2v[:, 0].reshape(2500, 128))
    s1 = _total_sum(edge_attr_v2v[:, 1].reshape(2500, 128))
    s = jnp.concatenate([s0, s1], axis=1)

    w1v = jnp.concatenate([p['c1_v2v']['Wl'], p['c1_v2v']['Wr'],
                           p['c1_i2v']['Wr'], p['c1_v2i']['Wl']], axis=1)
    b1v = jnp.concatenate([p['c1_v2v']['bl'], p['c1_v2v']['br'],
                           p['c1_i2v']['br'], p['c1_v2i']['bl']])
    xl_v2v, xr_v2v, xr_i2v, xl_v2i = _mm(xv, w1v, b1v, 2048)
    w1r = jnp.concatenate([p['c1_i2v']['Wl'], p['c1_v2i']['Wr']], axis=1)
    b1r = jnp.concatenate([p['c1_i2v']['bl'], p['c1_v2i']['br']])
    xl_i2v, xr_v2i = _mm(xu, w1r, b1r, 512)

    def _cat(num, den):
        n = jnp.concatenate([num[0, :_NHALF], num[1, :_NHALF]], axis=0)
        d2 = den.reshape(2, _ACC_VL, 2)
        d = jnp.concatenate([d2[0, :_NHALF], d2[1, :_NHALF]], axis=0)
        return n, d

    nv1, dv1 = _cat(*_edge_pass(2, _ACC_VL, epv, True, _NHALF)(
        xl_v2v, xr_v2v, edv, _wmat(p['c1_v2v'])))
    ni1, di1 = _cat(*_edge_pass(2, _ACC_VL, epi, True, _NHALF)(
        xl_i2v, xr_i2v, edi, _wmat(p['c1_i2v'])))
    nb1, db1 = _edge_pass(2, _ACC_R, epb, False)(
        xl_v2i, xr_v2i, edb, _wmat(p['c1_v2i']))

    v1 = _combine1_veh(nv1, dv1, ni1, di1, xl_v2v, xr_v2v,
                       _wmat(p['c1_v2v']), s,
                       p['c1_v2v']['bias'].reshape(1, -1),
                       p['c1_i2v']['bias'].reshape(1, -1))
    r1 = _combine1_rsu(nb1, db1.reshape(2, _ACC_R, 2),
                       p['c1_v2i']['bias'].reshape(1, -1))

    w2v = jnp.concatenate([p['c2_v2v']['Wl'], p['c2_v2v']['Wr'],
                           p['c2_i2v']['Wr'], p['c2_v2i']['Wl']], axis=1)
    b2v = jnp.concatenate([p['c2_v2v']['bl'], p['c2_v2v']['br'],
                           p['c2_i2v']['br'], p['c2_v2i']['bl']])
    xl2_v2v, xr2_v2v, xr2_i2v, xl2_v2i = _mm(v1, w2v, b2v, 2048)
    w2r = jnp.concatenate([p['c2_i2v']['Wl'], p['c2_v2i']['Wr']], axis=1)
    b2r = jnp.concatenate([p['c2_i2v']['bl'], p['c2_v2i']['br']])
    xl2_i2v, xr2_v2i = _mm(r1, w2r, b2r, 512)

    nv2, dv2 = _cat(*_edge_pass(1, _ACC_VL, epv, True, _NHALF)(
        xl2_v2v, xr2_v2v, edv, _wmat(p['c2_v2v'])))
    ni2, di2 = _cat(*_edge_pass(1, _ACC_VL, epi, True, _NHALF)(
        xl2_i2v, xr2_i2v, edi, _wmat(p['c2_i2v'])))
    nb2, db2 = _edge_pass(1, _ACC_R, epb, False)(
        xl2_v2i, xr2_v2i, edb, _wmat(p['c2_v2i']))

    v2 = _combine2_veh(nv2, dv2, ni2, di2, xl2_v2v, xr2_v2v,
                       _wmat(p['c2_v2v']), s,
                       p['c2_v2v']['bias'].reshape(1, -1),
                       p['c2_i2v']['bias'].reshape(1, -1),
                       p['ln_veh_g'].reshape(1, -1), p['ln_veh_b'].reshape(1, -1))
    r2 = _combine2_rsu(nb2, db2.reshape(2, _ACC_R, 2),
                       p['c2_v2i']['bias'].reshape(1, -1),
                       p['ln_rsu_g'].reshape(1, -1), p['ln_rsu_b'].reshape(1, -1))
    return (v2[:_NV], r2[:_NR])


# speedup vs baseline: 1.0197x; 1.0197x over previous
import dataclasses
import functools

import jax
import jax.numpy as jnp
from jax import lax
from jax.experimental import pallas as pl
from jax.experimental.pallas import tpu as pltpu
from jax.experimental.pallas import tpu_sc as plsc

_NV, _NR = 10000, 500
_EV = 320000
_NVP = 10240
_NRP = 512
_NHALF = 5120
_ACC_VL = 6144
_ACC_R = 2048
_C = 96
_ZR = 64
_EPS = 1e-16

_SC_PARAMS = pltpu.CompilerParams()
if "needs_layout_passes" in pltpu.CompilerParams.__dataclass_fields__:
    _SC_PARAMS = dataclasses.replace(_SC_PARAMS, needs_layout_passes=False)


def _edge_pass(heads, n_dst_pad, e_pad, split_dst, n_half=0):
    nworkers = 16 if split_dst else 32
    ept = e_pad // nworkers
    nchunks = ept // _C
    rows_sub = n_dst_pad // 16
    ncopies = rows_sub // _ZR
    nd2r = (n_dst_pad * 2) // 128
    mesh = plsc.VectorSubcoreMesh(core_axis_name="c", subcore_axis_name="s",
                                  num_cores=2, num_subcores=16)

    def body(xl_hbm, xr_hbm, ed_hbm, w_hbm, num_hbm, den_hbm,
             ed0, ed1, src0, src1, dstg0, dstg1, dst20, dst21,
             ea00, ea01, ea10, ea11, w_v, xl0, xl1, xr0, xr1, msg_v, den_t,
             idx_r, acc_sh, accd_sh,
             semi0, semi1, semg0, semg1):
        cid = lax.axis_index("c")
        sid = lax.axis_index("s")
        wid = sid if split_dst else cid * 16 + sid
        lane = lax.iota(jnp.int32, 16)
        lane01 = jnp.minimum(lane, 1)
        zeros = jnp.zeros((16,), jnp.float32)
        ed = [ed0, ed1]
        src = [src0, src1]
        dstg = [dstg0, dstg1]
        dst2 = [dst20, dst21]
        ea0 = [ea00, ea01]
        ea1 = [ea10, ea11]
        xl = [xl0, xl1]
        xr = [xr0, xr1]
        semi = [semi0, semi1]
        semg = [semg0, semg1]

        @pl.loop(0, _C)
        def _z(i):
            for j in range(8):
                msg_v[i, pl.ds(j * 16, 16)] = zeros

        @pl.loop(0, ncopies)
        def _za(k):
            r0 = sid * rows_sub + k * _ZR
            pltpu.sync_copy(msg_v.at[pl.ds(0, _ZR)], acc_sh.at[pl.ds(r0, _ZR)])

        @pl.when(sid == 0)
        def _zd():
            off = 0
            while off < nd2r:
                sz = min(_ZR, nd2r - off)
                pltpu.sync_copy(msg_v.at[pl.ds(0, sz)], accd_sh.at[pl.ds(off, sz)])
                off += sz

        @pl.loop(0, nd2r)
        def _zt(i):
            for j in range(8):
                den_t[i, pl.ds(j * 16, 16)] = zeros

        @pl.loop(0, nd2r // 16)
        def _zi(k):
            idx_r[pl.ds(k * 16, 16)] = lane + k * 16

        pltpu.sync_copy(w_hbm, w_v)
        plsc.subcore_barrier()

        att = [w_v[0, pl.ds(j * 16, 16)] for j in range(8)]
        we0 = [w_v[1, pl.ds(j * 16, 16)] for j in range(8)]
        we1 = [w_v[2, pl.ds(j * 16, 16)] for j in range(8)]

        def fetch_ed(i, b):
            blk = jnp.minimum(wid * nchunks + i, (wid + 1) * nchunks - 1)
            pltpu.async_copy(ed_hbm.at[blk], ed[b], semi[b])

        def wait_ed(b):
            pltpu.make_async_copy(ed_hbm.at[0], ed[b], semi[b]).wait()

        def prep(b):
            @pl.loop(0, _C // 16)
            def _p(g):
                sl = pl.ds(g * 16, 16)
                src[b][sl] = ed[b][0, sl]
                d = ed[b][1, sl]
                dstg[b][sl] = d
                ea0[b][sl] = plsc.bitcast(ed[b][2, sl], jnp.float32)
                ea1[b][sl] = plsc.bitcast(ed[b][3, sl], jnp.float32)
                if split_dst:
                    dl = d - cid * n_half
                    ok = (dl >= 0) & (dl < n_half)
                    dst2[b][sl] = jnp.where(ok, dl, n_half)
                else:
                    dst2[b][sl] = d

        def fire_gathers(b):
            d1 = pltpu.async_copy(xl_hbm.at[src[b]], xl[b], semg[b])
            d2 = pltpu.async_copy(xr_hbm.at[dstg[b]], xr[b], semg[b])
            return d1, d2

        def compute(b):
            @pl.loop(0, _C)
            def _edge(e):
                ei = jnp.full((16,), e, jnp.int32)
                a0 = plsc.load_gather(ea0[b], [ei])
                a1 = plsc.load_gather(ea1[b], [ei])
                dstv = plsc.load_gather(dst2[b], [ei])
                xls = []
                s0 = zeros
                s1 = zeros
                for j in range(8):
                    xlj = xl[b][e, pl.ds(j * 16, 16)]
                    xrj = xr[b][e, pl.ds(j * 16, 16)]
                    m = xlj + xrj + a0 * we0[j] + a1 * we1[j]
                    m = jnp.maximum(m, 0.2 * m)
                    if heads == 2 and j >= 4:
                        s1 = s1 + m * att[j]
                    else:
                        s0 = s0 + m * att[j]
                    xls.append(xlj)
                if heads == 2:
                    e0 = jnp.exp(jnp.full((16,), jnp.sum(s0), jnp.float32))
                    e1 = jnp.exp(jnp.full((16,), jnp.sum(s1), jnp.float32))
                    for j in range(8):
                        msg_v[e, pl.ds(j * 16, 16)] = xls[j] * (e0 if j < 4 else e1)
                    den = jnp.where(lane == 0, e0, jnp.where(lane == 1, e1, zeros))
                else:
                    e0 = jnp.exp(jnp.full((16,), jnp.sum(s0), jnp.float32))
                    for j in range(8):
                        msg_v[e, pl.ds(j * 16, 16)] = xls[j] * e0
                    den = jnp.where(lane == 0, e0, zeros)
                flat = dstv * 2 + lane01
                plsc.addupdate_scatter(
                    den_t,
                    [lax.shift_right_logical(flat, 7),
                     lax.bitwise_and(flat, 127)],
                    den, mask=lane < heads)

        fetch_ed(0, 0)
        fetch_ed(1, 1)

        @pl.loop(0, nchunks // 2)
        def _pair(tt):
            i = tt * 2
            wait_ed(0)
            prep(0)
            da = fire_gathers(0)
            wait_ed(1)
            prep(1)
            db = fire_gathers(1)
            da[0].wait()
            da[1].wait()
            compute(0)
            pltpu.sync_copy(msg_v, acc_sh.at[dst2[0]], add=True)
            fetch_ed(i + 2, 0)
            db[0].wait()
            db[1].wait()
            compute(1)
            pltpu.sync_copy(msg_v, acc_sh.at[dst2[1]], add=True)
            fetch_ed(i + 3, 1)

        wait_ed(0)
        wait_ed(1)

        pltpu.sync_copy(den_t, accd_sh.at[idx_r], add=True)
        plsc.subcore_barrier()

        @pl.loop(0, ncopies)
        def _wb(k):
            r0 = sid * rows_sub + k * _ZR
            pltpu.sync_copy(acc_sh.at[pl.ds(r0, _ZR)], num_hbm.at[cid, pl.ds(r0, _ZR)])

        @pl.when(sid == 0)
        def _wbd():
            pltpu.sync_copy(accd_sh, den_hbm.at[cid])

    return pl.kernel(
        body,
        out_type=(jax.ShapeDtypeStruct((2, n_dst_pad, 128), jnp.float32),
                  jax.ShapeDtypeStruct((2, nd2r, 128), jnp.float32)),
        mesh=mesh,
        compiler_params=_SC_PARAMS,
        scratch_types=(
            [pltpu.VMEM((4, _C), jnp.int32)] * 2
            + [pltpu.VMEM((_C,), jnp.int32)] * 6
            + [pltpu.VMEM((_C,), jnp.float32)] * 4
            + [pltpu.VMEM((3, 128), jnp.float32)]
            + [pltpu.VMEM((_C, 128), jnp.float32)] * 5
            + [pltpu.VMEM((nd2r, 128), jnp.float32),
               pltpu.VMEM((nd2r,), jnp.int32),
               pltpu.VMEM_SHARED((n_dst_pad, 128), jnp.float32),
               pltpu.VMEM_SHARED((nd2r, 128), jnp.float32)]
            + [pltpu.SemaphoreType.DMA] * 4
        ),
    )


def _mm(x, w, b, bm):
    m, k = x.shape
    n = w.shape[1]
    nout = n // 128

    def body(x_ref, w_ref, b_ref, *outs):
        y = jnp.dot(x_ref[...], w_ref[...], preferred_element_type=jnp.float32)
        y = y + b_ref[...]
        for i, o in enumerate(outs):
            o[...] = y[:, i * 128:(i + 1) * 128]

    return pl.pallas_call(
        body,
        grid=(m // bm,),
        in_specs=[
            pl.BlockSpec((bm, k), lambda i: (i, 0)),
            pl.BlockSpec((k, n), lambda i: (0, 0)),
            pl.BlockSpec((1, n), lambda i: (0, 0)),
        ],
        out_specs=[pl.BlockSpec((bm, 128), lambda i: (i, 0)) for _ in range(nout)],
        out_shape=[jax.ShapeDtypeStruct((m, 128), jnp.float32) for _ in range(nout)],
    )(x, w, b.reshape(1, n))


def _total_sum(x2d):
    def body(x_ref, o_ref):
        o_ref[...] = jnp.sum(x_ref[...])[None, None]

    return pl.pallas_call(
        body, out_shape=jax.ShapeDtypeStruct((1, 1), jnp.float32))(x2d)


def _combine1_veh(numv, denv, numi, deni, xl, xr, w, s, bias_v, bias_i):
    bm = 1024

    def body(nv_ref, dv_ref, ni_ref, di_ref, xl_ref, xr_ref, w_ref, s_ref,
             bv_ref, bi_ref, o_ref):
        xl_ = xl_ref[...]
        w_ = w_ref[...]
        xe = (s_ref[0, 0] / _EV) * w_[1:2, :] + (s_ref[0, 1] / _EV) * w_[2:3, :]
        m = xl_ + xr_ref[...] + xe
        m = jnp.maximum(m, 0.2 * m)
        am = m * w_[0:1, :]
        e0 = jnp.exp(jnp.sum(am[:, :64], axis=1, keepdims=True))
        e1 = jnp.exp(jnp.sum(am[:, 64:], axis=1, keepdims=True))
        av = nv_ref[...]
        dv = dv_ref[...]
        num = av + jnp.concatenate([e0 * xl_[:, :64], e1 * xl_[:, 64:]], axis=1)
        ov = jnp.concatenate(
            [num[:, :64] / (dv[:, 0:1] + e0 + _EPS),
             num[:, 64:] / (dv[:, 1:2] + e1 + _EPS)], axis=1) + bv_ref[...]
        ai = ni_ref[...]
        di = di_ref[...]
        oi = jnp.concatenate(
            [ai[:, :64] / (di[:, 0:1] + _EPS),
             ai[:, 64:] / (di[:, 1:2] + _EPS)], axis=1) + bi_ref[...]
        v = ov + oi
        o_ref[...] = jnp.where(v > 0, v, jnp.exp(v) - 1.0)

    return pl.pallas_call(
        body,
        grid=(_NVP // bm,),
        in_specs=[
            pl.BlockSpec((bm, 128), lambda i: (i, 0)),
            pl.BlockSpec((bm, 2), lambda i: (i, 0)),
            pl.BlockSpec((bm, 128), lambda i: (i, 0)),
            pl.BlockSpec((bm, 2), lambda i: (i, 0)),
            pl.BlockSpec((bm, 128), lambda i: (i, 0)),
            pl.BlockSpec((bm, 128), lambda i: (i, 0)),
            pl.BlockSpec((3, 128), lambda i: (0, 0)),
            pl.BlockSpec(memory_space=pltpu.SMEM),
            pl.BlockSpec((1, 128), lambda i: (0, 0)),
            pl.BlockSpec((1, 128), lambda i: (0, 0)),
        ],
        out_specs=pl.BlockSpec((bm, 128), lambda i: (i, 0)),
        out_shape=jax.ShapeDtypeStruct((_NVP, 128), jnp.float32),
    )(numv, denv, numi, deni, xl, xr, w, s, bias_v, bias_i)


def _combine1_rsu(num, den, bias):
    def body(n_ref, d_ref, b_ref, o_ref):
        a = n_ref[0] + n_ref[1]
        d = d_ref[0] + d_ref[1]
        o = jnp.concatenate(
            [a[:, :64] / (d[:, 0:1] + _EPS),
             a[:, 64:] / (d[:, 1:2] + _EPS)], axis=1) + b_ref[...]
        o_ref[...] = jnp.where(o > 0, o, jnp.exp(o) - 1.0)

    return pl.pallas_call(
        body,
        grid=(1,),
        in_specs=[
            pl.BlockSpec((2, _NRP, 128), lambda i: (0, 0, 0)),
            pl.BlockSpec((2, _NRP, 2), lambda i: (0, 0, 0)),
            pl.BlockSpec((1, 128), lambda i: (0, 0)),
        ],
        out_specs=pl.BlockSpec((_NRP, 128), lambda i: (0, 0)),
        out_shape=jax.ShapeDtypeStruct((_NRP, 128), jnp.float32),
    )(num, den, bias)


def _combine2_veh(numv, denv, numi, deni, xl, xr, w, s, bias_v, bias_i, g, b):
    bm = 1024

    def body(nv_ref, dv_ref, ni_ref, di_ref, xl_ref, xr_ref, w_ref, s_ref,
             bv_ref, bi_ref, g_ref, be_ref, o_ref):
        xl_ = xl_ref[...]
        w_ = w_ref[...]
        xe = (s_ref[0, 0] / _EV) * w_[1:2, :] + (s_ref[0, 1] / _EV) * w_[2:3, :]
        m = xl_ + xr_ref[...] + xe
        m = jnp.maximum(m, 0.2 * m)
        e = jnp.exp(jnp.sum(m * w_[0:1, :], axis=1, keepdims=True))
        av = nv_ref[...]
        dv = dv_ref[...]
        ov = (av + e * xl_) / (dv[:, 0:1] + e + _EPS) + bv_ref[...]
        ai = ni_ref[...]
        di = di_ref[...]
        oi = ai / (di[:, 0:1] + _EPS) + bi_ref[...]
        v = ov + oi
        mu = jnp.mean(v, axis=1, keepdims=True)
        cv = v - mu
        var = jnp.mean(cv * cv, axis=1, keepdims=True)
        o_ref[...] = cv * lax.rsqrt(var + 1e-5) * g_ref[...] + be_ref[...]

    return pl.pallas_call(
        body,
        grid=(_NVP // bm,),
        in_specs=[
            pl.BlockSpec((bm, 128), lambda i: (i, 0)),
            pl.BlockSpec((bm, 2), lambda i: (i, 0)),
            pl.BlockSpec((bm, 128), lambda i: (i, 0)),
            pl.BlockSpec((bm, 2), lambda i: (i, 0)),
            pl.BlockSpec((bm, 128), lambda i: (i, 0)),
            pl.BlockSpec((bm, 128), lambda i: (i, 0)),
            pl.BlockSpec((3, 128), lambda i: (0, 0)),
            pl.BlockSpec(memory_space=pltpu.SMEM),
            pl.BlockSpec((1, 128), lambda i: (0, 0)),
            pl.BlockSpec((1, 128), lambda i: (0, 0)),
            pl.BlockSpec((1, 128), lambda i: (0, 0)),
            pl.BlockSpec((1, 128), lambda i: (0, 0)),
        ],
        out_specs=pl.BlockSpec((bm, 128), lambda i: (i, 0)),
        out_shape=jax.ShapeDtypeStruct((_NVP, 128), jnp.float32),
    )(numv, denv, numi, deni, xl, xr, w, s, bias_v, bias_i, g, b)


def _combine2_rsu(num, den, bias, g, b):
    def body(n_ref, d_ref, b_ref, g_ref, be_ref, o_ref):
        a = n_ref[0] + n_ref[1]
        d = d_ref[0] + d_ref[1]
        v = a / (d[:, 0:1] + _EPS) + b_ref[...]
        mu = jnp.mean(v, axis=1, keepdims=True)
        cv = v - mu
        var = jnp.mean(cv * cv, axis=1, keepdims=True)
        o_ref[...] = cv * lax.rsqrt(var + 1e-5) * g_ref[...] + be_ref[...]

    return pl.pallas_call(
        body,
        grid=(1,),
        in_specs=[
            pl.BlockSpec((2, _NRP, 128), lambda i: (0, 0, 0)),
            pl.BlockSpec((2, _NRP, 2), lambda i: (0, 0, 0)),
            pl.BlockSpec((1, 128), lambda i: (0, 0)),
            pl.BlockSpec((1, 128), lambda i: (0, 0)),
            pl.BlockSpec((1, 128), lambda i: (0, 0)),
        ],
        out_specs=pl.BlockSpec((_NRP, 128), lambda i: (0, 0)),
        out_shape=jax.ShapeDtypeStruct((_NRP, 128), jnp.float32),
    )(num, den, bias, g, b)


def _pad_edges(src, dst, ea, n_dst):
    e = src.shape[0]
    ep = -(-e // 6144) * 6144
    pad = ep - e
    src_p = jnp.concatenate([src.astype(jnp.int32), jnp.zeros((pad,), jnp.int32)])
    dst_p = jnp.concatenate([dst.astype(jnp.int32),
                             jnp.full((pad,), n_dst, jnp.int32)])
    ea0 = jnp.concatenate([ea[:, 0], jnp.zeros((pad,), jnp.float32)])
    ea1 = jnp.concatenate([ea[:, 1], jnp.zeros((pad,), jnp.float32)])
    edata = jnp.stack([src_p.reshape(-1, _C),
                       dst_p.reshape(-1, _C),
                       lax.bitcast_convert_type(ea0, jnp.int32).reshape(-1, _C),
                       lax.bitcast_convert_type(ea1, jnp.int32).reshape(-1, _C)],
                      axis=1)
    return edata, ep


def _wmat(cp):
    return jnp.stack([cp['att'].reshape(-1), cp['We'][0], cp['We'][1]])


def kernel(x_vehicle, x_rsu, edge_index_v2v, v2i_src, v2i_dst, i2v_src, i2v_dst,
           edge_attr_v2v, edge_attr_v2i, edge_attr_i2v, params):
    p = params
    xv = jnp.pad(x_vehicle, ((0, _NVP - _NV), (0, 0)))
    xu = jnp.pad(x_rsu, ((0, _NRP - _NR), (0, 0)))
    sv, dv = edge_index_v2v[0], edge_index_v2v[1]

    edv, epv = _pad_edges(sv, dv, edge_attr_v2v, _NV)
    edi, epi = _pad_edges(i2v_src, i2v_dst, edge_attr_i2v, _NV)
    edb, epb = _pad_edges(v2i_src, v2i_dst, edge_attr_v2i, _NR)

    s0 = _total_sum(edge_attr_v2v[:, 0].reshape(2500, 128))
    s1 = _total_sum(edge_attr_v2v[:, 1].reshape(2500, 128))
    s = jnp.concatenate([s0, s1], axis=1)

    w1v = jnp.concatenate([p['c1_v2v']['Wl'], p['c1_v2v']['Wr'],
                           p['c1_i2v']['Wr'], p['c1_v2i']['Wl']], axis=1)
    b1v = jnp.concatenate([p['c1_v2v']['bl'], p['c1_v2v']['br'],
                           p['c1_i2v']['br'], p['c1_v2i']['bl']])
    xl_v2v, xr_v2v, xr_i2v, xl_v2i = _mm(xv, w1v, b1v, 2048)
    w1r = jnp.concatenate([p['c1_i2v']['Wl'], p['c1_v2i']['Wr']], axis=1)
    b1r = jnp.concatenate([p['c1_i2v']['bl'], p['c1_v2i']['br']])
    xl_i2v, xr_v2i = _mm(xu, w1r, b1r, 512)

    def _cat(num, den):
        n = jnp.concatenate([num[0, :_NHALF], num[1, :_NHALF]], axis=0)
        d2 = den.reshape(2, _ACC_VL, 2)
        d = jnp.concatenate([d2[0, :_NHALF], d2[1, :_NHALF]], axis=0)
        return n, d

    nv1, dv1 = _cat(*_edge_pass(2, _ACC_VL, epv, True, _NHALF)(
        xl_v2v, xr_v2v, edv, _wmat(p['c1_v2v'])))
    ni1, di1 = _cat(*_edge_pass(2, _ACC_VL, epi, True, _NHALF)(
        xl_i2v, xr_i2v, edi, _wmat(p['c1_i2v'])))
    nb1, db1 = _edge_pass(2, _ACC_R, epb, False)(
        xl_v2i, xr_v2i, edb, _wmat(p['c1_v2i']))

    v1 = _combine1_veh(nv1, dv1, ni1, di1, xl_v2v, xr_v2v,
                       _wmat(p['c1_v2v']), s,
                       p['c1_v2v']['bias'].reshape(1, -1),
                       p['c1_i2v']['bias'].reshape(1, -1))
    r1 = _combine1_rsu(nb1, db1.reshape(2, _ACC_R, 2),
                       p['c1_v2i']['bias'].reshape(1, -1))

    w2v = jnp.concatenate([p['c2_v2v']['Wl'], p['c2_v2v']['Wr'],
                           p['c2_i2v']['Wr'], p['c2_v2i']['Wl']], axis=1)
    b2v = jnp.concatenate([p['c2_v2v']['bl'], p['c2_v2v']['br'],
                           p['c2_i2v']['br'], p['c2_v2i']['bl']])
    xl2_v2v, xr2_v2v, xr2_i2v, xl2_v2i = _mm(v1, w2v, b2v, 2048)
    w2r = jnp.concatenate([p['c2_i2v']['Wl'], p['c2_v2i']['Wr']], axis=1)
    b2r = jnp.concatenate([p['c2_i2v']['bl'], p['c2_v2i']['br']])
    xl2_i2v, xr2_v2i = _mm(r1, w2r, b2r, 512)

    nv2, dv2 = _cat(*_edge_pass(1, _ACC_VL, epv, True, _NHALF)(
        xl2_v2v, xr2_v2v, edv, _wmat(p['c2_v2v'])))
    ni2, di2 = _cat(*_edge_pass(1, _ACC_VL, epi, True, _NHALF)(
        xl2_i2v, xr2_i2v, edi, _wmat(p['c2_i2v'])))
    nb2, db2 = _edge_pass(1, _ACC_R, epb, False)(
        xl2_v2i, xr2_v2i, edb, _wmat(p['c2_v2i']))

    v2 = _combine2_veh(nv2, dv2, ni2, di2, xl2_v2v, xr2_v2v,
                       _wmat(p['c2_v2v']), s,
                       p['c2_v2v']['bias'].reshape(1, -1),
                       p['c2_i2v']['bias'].reshape(1, -1),
                       p['ln_veh_g'].reshape(1, -1), p['ln_veh_b'].reshape(1, -1))
    r2 = _combine2_rsu(nb2, db2.reshape(2, _ACC_R, 2),
                       p['c2_v2i']['bias'].reshape(1, -1),
                       p['ln_rsu_g'].reshape(1, -1), p['ln_rsu_b'].reshape(1, -1))
    return (v2[:_NV], r2[:_NR])


# speedup vs baseline: 1.2411x; 1.2172x over previous
import dataclasses
import functools

import jax
import jax.numpy as jnp
from jax import lax
from jax.experimental import pallas as pl
from jax.experimental.pallas import tpu as pltpu
from jax.experimental.pallas import tpu_sc as plsc

_NV, _NR = 10000, 500
_EV = 320000
_NVP = 10240
_NRP = 512
_NHALF = 5120
_ACC_VL = 6144
_ACC_R = 2048
_C = 96
_P = 208
_ZR = 64
_EPS = 1e-16

_SC_PARAMS = pltpu.CompilerParams()
if "needs_layout_passes" in pltpu.CompilerParams.__dataclass_fields__:
    _SC_PARAMS = dataclasses.replace(_SC_PARAMS, needs_layout_passes=False)


def _edge_pass(heads, n_dst_pad, e_pad, split_dst, n_half=0):
    nworkers = 16 if split_dst else 32
    ept = e_pad // nworkers
    nchunks = ept // _C
    rows_sub = n_dst_pad // 16
    ncopies = rows_sub // _ZR
    nd2r = (n_dst_pad * 2) // 128
    mesh = plsc.VectorSubcoreMesh(core_axis_name="c", subcore_axis_name="s",
                                  num_cores=2, num_subcores=16)

    def body(xl_hbm, xr_hbm, ed_hbm, w_hbm, num_hbm, den_hbm,
             ed0, ed1, src0, src1, dstg0, dstg1, dst20, dst21,
             ea00, ea01, ea10, ea11, w_v, xl0, xl1, xr0, xr1, msg_v, den_t,
             idx_r, acc_sh, accd_sh,
             semi0, semi1, semg0, semg1):
        cid = lax.axis_index("c")
        sid = lax.axis_index("s")
        wid = sid if split_dst else cid * 16 + sid
        lane = lax.iota(jnp.int32, 16)
        lane01 = jnp.minimum(lane, 1)
        zeros = jnp.zeros((16,), jnp.float32)
        ed = [ed0, ed1]
        src = [src0, src1]
        dstg = [dstg0, dstg1]
        dst2 = [dst20, dst21]
        ea0 = [ea00, ea01]
        ea1 = [ea10, ea11]
        xl = [xl0, xl1]
        xr = [xr0, xr1]
        semi = [semi0, semi1]
        semg = [semg0, semg1]

        @pl.loop(0, _C)
        def _z(i):
            for j in range(8):
                msg_v[i, pl.ds(j * 16, 16)] = zeros

        @pl.loop(0, ncopies)
        def _za(k):
            r0 = sid * rows_sub + k * _ZR
            pltpu.sync_copy(msg_v.at[pl.ds(0, _ZR)], acc_sh.at[pl.ds(r0, _ZR)])

        @pl.when(sid == 0)
        def _zd():
            off = 0
            while off < nd2r:
                sz = min(_ZR, nd2r - off)
                pltpu.sync_copy(msg_v.at[pl.ds(0, sz)], accd_sh.at[pl.ds(off, sz)])
                off += sz

        @pl.loop(0, nd2r)
        def _zt(i):
            for j in range(8):
                den_t[i, pl.ds(j * 16, 16)] = zeros

        @pl.loop(0, nd2r // 16)
        def _zi(k):
            idx_r[pl.ds(k * 16, 16)] = lane + k * 16

        pltpu.sync_copy(w_hbm, w_v)
        plsc.subcore_barrier()

        att = [w_v[0, pl.ds(j * 16, 16)] for j in range(8)]
        we0 = [w_v[1, pl.ds(j * 16, 16)] for j in range(8)]
        we1 = [w_v[2, pl.ds(j * 16, 16)] for j in range(8)]

        def fetch_ed(i, b):
            blk = jnp.minimum(wid * nchunks + i, (wid + 1) * nchunks - 1)
            pltpu.async_copy(ed_hbm.at[blk], ed[b], semi[b])

        def wait_ed(b):
            pltpu.make_async_copy(ed_hbm.at[0], ed[b], semi[b]).wait()

        def compute(ea0_r, ea1_r, dst_r, xl_r, xr_r):
            @pl.loop(0, _C)
            def _edge(e):
                ei = jnp.full((16,), e, jnp.int32)
                a0 = plsc.bitcast(plsc.load_gather(ea0_r, [ei]), jnp.float32)
                a1 = plsc.bitcast(plsc.load_gather(ea1_r, [ei]), jnp.float32)
                dstv = plsc.load_gather(dst_r, [ei])
                xls = []
                s0 = zeros
                s1 = zeros
                for j in range(8):
                    xlj = xl_r[e, pl.ds(j * 16, 16)]
                    xrj = xr_r[e, pl.ds(j * 16, 16)]
                    m = xlj + xrj + a0 * we0[j] + a1 * we1[j]
                    m = jnp.maximum(m, 0.2 * m)
                    if heads == 2 and j >= 4:
                        s1 = s1 + m * att[j]
                    else:
                        s0 = s0 + m * att[j]
                    xls.append(xlj)
                if heads == 2:
                    e0 = jnp.exp(jnp.full((16,), jnp.sum(s0), jnp.float32))
                    e1 = jnp.exp(jnp.full((16,), jnp.sum(s1), jnp.float32))
                    for j in range(8):
                        msg_v[e, pl.ds(j * 16, 16)] = xls[j] * (e0 if j < 4 else e1)
                    den = jnp.where(lane == 0, e0, jnp.where(lane == 1, e1, zeros))
                else:
                    e0 = jnp.exp(jnp.full((16,), jnp.sum(s0), jnp.float32))
                    for j in range(8):
                        msg_v[e, pl.ds(j * 16, 16)] = xls[j] * e0
                    den = jnp.where(lane == 0, e0, zeros)
                flat = dstv * 2 + lane01
                plsc.addupdate_scatter(
                    den_t,
                    [lax.shift_right_logical(flat, 7),
                     lax.bitwise_and(flat, 127)],
                    den, mask=lane < heads)

        if split_dst:
            pend = [src0, dstg0, ea00, ea10]

            def process():
                @pl.loop(0, _C // 16)
                def _cp(g):
                    sl = pl.ds(g * 16, 16)
                    dst20[sl] = dstg0[sl]
                d1 = pltpu.async_copy(xl_hbm.at[src0.at[pl.ds(0, _C)]],
                                      xl0, semg0)
                d2 = pltpu.async_copy(xr_hbm.at[dst21.at[pl.ds(0, _C)]],
                                      xr0, semg1)
                d1.wait()
                d2.wait()
                compute(ea00, ea10, dst20, xl0, xr0)
                pltpu.sync_copy(msg_v, acc_sh.at[dst20], add=True)
                for arr in pend + [dst21]:
                    for g in range(7):
                        sl = pl.ds(g * 16, 16)
                        arr[sl] = arr[pl.ds(_C + g * 16, 16)]

            fetch_ed(0, 0)
            fetch_ed(1, 1)

            def _pair(tt, fill):
                for b in (0, 1):
                    i = tt * 2 + b
                    wait_ed(b)
                    for g in range(_C // 16):
                        sl = pl.ds(g * 16, 16)
                        d_ = ed[b][1, sl]
                        dl = d_ - cid * n_half
                        ok = (dl >= 0) & (dl < n_half)
                        at = pl.ds(fill, 16)
                        plsc.store_compressed(src0.at[at], ed[b][0, sl], mask=ok)
                        plsc.store_compressed(dstg0.at[at], dl, mask=ok)
                        plsc.store_compressed(dst21.at[at], d_, mask=ok)
                        plsc.store_compressed(ea00.at[at], ed[b][2, sl], mask=ok)
                        plsc.store_compressed(ea10.at[at], ed[b][3, sl], mask=ok)
                        cnt = plsc.all_reduce_population_count(ok)
                        fill = fill + jnp.max(cnt)
                    fetch_ed(i + 2, b)

                    @pl.when(fill >= _C)
                    def _go():
                        process()

                    fill = fill - jnp.where(fill >= _C, _C, 0)
                return fill

            fill = lax.fori_loop(0, nchunks // 2, _pair, 0)

            fillv = jnp.full((16,), fill, jnp.int32)

            @pl.loop(0, _C // 16)
            def _pad(g):
                sl = pl.ds(g * 16, 16)
                gl = lane + g * 16
                padm = gl >= fillv
                src0[sl] = jnp.where(padm, 0, src0[sl])
                dstg0[sl] = jnp.where(padm, n_half, dstg0[sl])
                dst21[sl] = jnp.where(padm, 0, dst21[sl])
                ea00[sl] = jnp.where(padm, 0, ea00[sl])
                ea10[sl] = jnp.where(padm, 0, ea10[sl])

            process()
            wait_ed(0)
            wait_ed(1)
        else:
            def prep(b):
                @pl.loop(0, _C // 16)
                def _p(g):
                    sl = pl.ds(g * 16, 16)
                    src[b][sl] = ed[b][0, sl]
                    d = ed[b][1, sl]
                    dstg[b][sl] = d
                    ea0[b][sl] = ed[b][2, sl]
                    ea1[b][sl] = ed[b][3, sl]
                    dst2[b][sl] = d

            def fire_gathers(b):
                d1 = pltpu.async_copy(xl_hbm.at[src[b]], xl[b], semg[b])
                d2 = pltpu.async_copy(xr_hbm.at[dstg[b]], xr[b], semg[b])
                return d1, d2

            fetch_ed(0, 0)
            fetch_ed(1, 1)

            @pl.loop(0, nchunks // 2)
            def _pairs(tt):
                i = tt * 2
                wait_ed(0)
                prep(0)
                da = fire_gathers(0)
                wait_ed(1)
                prep(1)
                db = fire_gathers(1)
                da[0].wait()
                da[1].wait()
                compute(ea0[0], ea1[0], dst2[0], xl0, xr0)
                pltpu.sync_copy(msg_v, acc_sh.at[dst2[0]], add=True)
                fetch_ed(i + 2, 0)
                db[0].wait()
                db[1].wait()
                compute(ea0[1], ea1[1], dst2[1], xl1, xr1)
                pltpu.sync_copy(msg_v, acc_sh.at[dst2[1]], add=True)
                fetch_ed(i + 3, 1)

            wait_ed(0)
            wait_ed(1)

        pltpu.sync_copy(den_t, accd_sh.at[idx_r], add=True)
        plsc.subcore_barrier()

        @pl.loop(0, ncopies)
        def _wb(k):
            r0 = sid * rows_sub + k * _ZR
            pltpu.sync_copy(acc_sh.at[pl.ds(r0, _ZR)], num_hbm.at[cid, pl.ds(r0, _ZR)])

        @pl.when(sid == 0)
        def _wbd():
            pltpu.sync_copy(accd_sh, den_hbm.at[cid])

    return pl.kernel(
        body,
        out_type=(jax.ShapeDtypeStruct((2, n_dst_pad, 128), jnp.float32),
                  jax.ShapeDtypeStruct((2, nd2r, 128), jnp.float32)),
        mesh=mesh,
        compiler_params=_SC_PARAMS,
        scratch_types=(
            [pltpu.VMEM((4, _C), jnp.int32)] * 2
            + ([pltpu.VMEM((_P,), jnp.int32),
                pltpu.VMEM((16,), jnp.int32),
                pltpu.VMEM((_P,), jnp.int32),
                pltpu.VMEM((16,), jnp.int32),
                pltpu.VMEM((_C,), jnp.int32),
                pltpu.VMEM((_P,), jnp.int32),
                pltpu.VMEM((_P,), jnp.int32),
                pltpu.VMEM((16,), jnp.int32),
                pltpu.VMEM((_P,), jnp.int32),
                pltpu.VMEM((16,), jnp.int32)]
               if split_dst else
               [pltpu.VMEM((_C,), jnp.int32)] * 10)
            + [pltpu.VMEM((3, 128), jnp.float32)]
            + ([pltpu.VMEM((_C, 128), jnp.float32),
                pltpu.VMEM((8, 128), jnp.float32),
                pltpu.VMEM((_C, 128), jnp.float32),
                pltpu.VMEM((8, 128), jnp.float32),
                pltpu.VMEM((_C, 128), jnp.float32)]
               if split_dst else
               [pltpu.VMEM((_C, 128), jnp.float32)] * 5)
            + [pltpu.VMEM((nd2r, 128), jnp.float32),
               pltpu.VMEM((nd2r,), jnp.int32),
               pltpu.VMEM_SHARED((n_dst_pad, 128), jnp.float32),
               pltpu.VMEM_SHARED((nd2r, 128), jnp.float32)]
            + [pltpu.SemaphoreType.DMA] * 4
        ),
    )


def _mm(x, w, b, bm):
    m, k = x.shape
    n = w.shape[1]
    nout = n // 128

    def body(x_ref, w_ref, b_ref, *outs):
        y = jnp.dot(x_ref[...], w_ref[...], preferred_element_type=jnp.float32)
        y = y + b_ref[...]
        for i, o in enumerate(outs):
            o[...] = y[:, i * 128:(i + 1) * 128]

    return pl.pallas_call(
        body,
        grid=(m // bm,),
        in_specs=[
            pl.BlockSpec((bm, k), lambda i: (i, 0)),
            pl.BlockSpec((k, n), lambda i: (0, 0)),
            pl.BlockSpec((1, n), lambda i: (0, 0)),
        ],
        out_specs=[pl.BlockSpec((bm, 128), lambda i: (i, 0)) for _ in range(nout)],
        out_shape=[jax.ShapeDtypeStruct((m, 128), jnp.float32) for _ in range(nout)],
    )(x, w, b.reshape(1, n))


def _total_sum(x2d):
    def body(x_ref, o_ref):
        o_ref[...] = jnp.sum(x_ref[...])[None, None]

    return pl.pallas_call(
        body, out_shape=jax.ShapeDtypeStruct((1, 1), jnp.float32))(x2d)


def _combine1_veh(numv, denv, numi, deni, xl, xr, w, s, bias_v, bias_i):
    bm = 1024

    def body(nv_ref, dv_ref, ni_ref, di_ref, xl_ref, xr_ref, w_ref, s_ref,
             bv_ref, bi_ref, o_ref):
        xl_ = xl_ref[...]
        w_ = w_ref[...]
        xe = (s_ref[0, 0] / _EV) * w_[1:2, :] + (s_ref[0, 1] / _EV) * w_[2:3, :]
        m = xl_ + xr_ref[...] + xe
        m = jnp.maximum(m, 0.2 * m)
        am = m * w_[0:1, :]
        e0 = jnp.exp(jnp.sum(am[:, :64], axis=1, keepdims=True))
        e1 = jnp.exp(jnp.sum(am[:, 64:], axis=1, keepdims=True))
        av = nv_ref[...]
        dv = dv_ref[...]
        num = av + jnp.concatenate([e0 * xl_[:, :64], e1 * xl_[:, 64:]], axis=1)
        ov = jnp.concatenate(
            [num[:, :64] / (dv[:, 0:1] + e0 + _EPS),
             num[:, 64:] / (dv[:, 1:2] + e1 + _EPS)], axis=1) + bv_ref[...]
        ai = ni_ref[...]
        di = di_ref[...]
        oi = jnp.concatenate(
            [ai[:, :64] / (di[:, 0:1] + _EPS),
             ai[:, 64:] / (di[:, 1:2] + _EPS)], axis=1) + bi_ref[...]
        v = ov + oi
        o_ref[...] = jnp.where(v > 0, v, jnp.exp(v) - 1.0)

    return pl.pallas_call(
        body,
        grid=(_NVP // bm,),
        in_specs=[
            pl.BlockSpec((bm, 128), lambda i: (i, 0)),
            pl.BlockSpec((bm, 2), lambda i: (i, 0)),
            pl.BlockSpec((bm, 128), lambda i: (i, 0)),
            pl.BlockSpec((bm, 2), lambda i: (i, 0)),
            pl.BlockSpec((bm, 128), lambda i: (i, 0)),
            pl.BlockSpec((bm, 128), lambda i: (i, 0)),
            pl.BlockSpec((3, 128), lambda i: (0, 0)),
            pl.BlockSpec(memory_space=pltpu.SMEM),
            pl.BlockSpec((1, 128), lambda i: (0, 0)),
            pl.BlockSpec((1, 128), lambda i: (0, 0)),
        ],
        out_specs=pl.BlockSpec((bm, 128), lambda i: (i, 0)),
        out_shape=jax.ShapeDtypeStruct((_NVP, 128), jnp.float32),
    )(numv, denv, numi, deni, xl, xr, w, s, bias_v, bias_i)


def _combine1_rsu(num, den, bias):
    def body(n_ref, d_ref, b_ref, o_ref):
        a = n_ref[0] + n_ref[1]
        d = d_ref[0] + d_ref[1]
        o = jnp.concatenate(
            [a[:, :64] / (d[:, 0:1] + _EPS),
             a[:, 64:] / (d[:, 1:2] + _EPS)], axis=1) + b_ref[...]
        o_ref[...] = jnp.where(o > 0, o, jnp.exp(o) - 1.0)

    return pl.pallas_call(
        body,
        grid=(1,),
        in_specs=[
            pl.BlockSpec((2, _NRP, 128), lambda i: (0, 0, 0)),
            pl.BlockSpec((2, _NRP, 2), lambda i: (0, 0, 0)),
            pl.BlockSpec((1, 128), lambda i: (0, 0)),
        ],
        out_specs=pl.BlockSpec((_NRP, 128), lambda i: (0, 0)),
        out_shape=jax.ShapeDtypeStruct((_NRP, 128), jnp.float32),
    )(num, den, bias)


def _combine2_veh(numv, denv, numi, deni, xl, xr, w, s, bias_v, bias_i, g, b):
    bm = 1024

    def body(nv_ref, dv_ref, ni_ref, di_ref, xl_ref, xr_ref, w_ref, s_ref,
             bv_ref, bi_ref, g_ref, be_ref, o_ref):
        xl_ = xl_ref[...]
        w_ = w_ref[...]
        xe = (s_ref[0, 0] / _EV) * w_[1:2, :] + (s_ref[0, 1] / _EV) * w_[2:3, :]
        m = xl_ + xr_ref[...] + xe
        m = jnp.maximum(m, 0.2 * m)
        e = jnp.exp(jnp.sum(m * w_[0:1, :], axis=1, keepdims=True))
        av = nv_ref[...]
        dv = dv_ref[...]
        ov = (av + e * xl_) / (dv[:, 0:1] + e + _EPS) + bv_ref[...]
        ai = ni_ref[...]
        di = di_ref[...]
        oi = ai / (di[:, 0:1] + _EPS) + bi_ref[...]
        v = ov + oi
        mu = jnp.mean(v, axis=1, keepdims=True)
        cv = v - mu
        var = jnp.mean(cv * cv, axis=1, keepdims=True)
        o_ref[...] = cv * lax.rsqrt(var + 1e-5) * g_ref[...] + be_ref[...]

    return pl.pallas_call(
        body,
        grid=(_NVP // bm,),
        in_specs=[
            pl.BlockSpec((bm, 128), lambda i: (i, 0)),
            pl.BlockSpec((bm, 2), lambda i: (i, 0)),
            pl.BlockSpec((bm, 128), lambda i: (i, 0)),
            pl.BlockSpec((bm, 2), lambda i: (i, 0)),
            pl.BlockSpec((bm, 128), lambda i: (i, 0)),
            pl.BlockSpec((bm, 128), lambda i: (i, 0)),
            pl.BlockSpec((3, 128), lambda i: (0, 0)),
            pl.BlockSpec(memory_space=pltpu.SMEM),
            pl.BlockSpec((1, 128), lambda i: (0, 0)),
            pl.BlockSpec((1, 128), lambda i: (0, 0)),
            pl.BlockSpec((1, 128), lambda i: (0, 0)),
            pl.BlockSpec((1, 128), lambda i: (0, 0)),
        ],
        out_specs=pl.BlockSpec((bm, 128), lambda i: (i, 0)),
        out_shape=jax.ShapeDtypeStruct((_NVP, 128), jnp.float32),
    )(numv, denv, numi, deni, xl, xr, w, s, bias_v, bias_i, g, b)


def _combine2_rsu(num, den, bias, g, b):
    def body(n_ref, d_ref, b_ref, g_ref, be_ref, o_ref):
        a = n_ref[0] + n_ref[1]
        d = d_ref[0] + d_ref[1]
        v = a / (d[:, 0:1] + _EPS) + b_ref[...]
        mu = jnp.mean(v, axis=1, keepdims=True)
        cv = v - mu
        var = jnp.mean(cv * cv, axis=1, keepdims=True)
        o_ref[...] = cv * lax.rsqrt(var + 1e-5) * g_ref[...] + be_ref[...]

    return pl.pallas_call(
        body,
        grid=(1,),
        in_specs=[
            pl.BlockSpec((2, _NRP, 128), lambda i: (0, 0, 0)),
            pl.BlockSpec((2, _NRP, 2), lambda i: (0, 0, 0)),
            pl.BlockSpec((1, 128), lambda i: (0, 0)),
            pl.BlockSpec((1, 128), lambda i: (0, 0)),
            pl.BlockSpec((1, 128), lambda i: (0, 0)),
        ],
        out_specs=pl.BlockSpec((_NRP, 128), lambda i: (0, 0)),
        out_shape=jax.ShapeDtypeStruct((_NRP, 128), jnp.float32),
    )(num, den, bias, g, b)


def _pad_edges(src, dst, ea, n_dst):
    e = src.shape[0]
    ep = -(-e // 6144) * 6144
    pad = ep - e
    src_p = jnp.concatenate([src.astype(jnp.int32), jnp.zeros((pad,), jnp.int32)])
    dst_p = jnp.concatenate([dst.astype(jnp.int32),
                             jnp.full((pad,), n_dst, jnp.int32)])
    ea0 = jnp.concatenate([ea[:, 0], jnp.zeros((pad,), jnp.float32)])
    ea1 = jnp.concatenate([ea[:, 1], jnp.zeros((pad,), jnp.float32)])
    edata = jnp.stack([src_p.reshape(-1, _C),
                       dst_p.reshape(-1, _C),
                       lax.bitcast_convert_type(ea0, jnp.int32).reshape(-1, _C),
                       lax.bitcast_convert_type(ea1, jnp.int32).reshape(-1, _C)],
                      axis=1)
    return edata, ep


def _wmat(cp):
    return jnp.stack([cp['att'].reshape(-1), cp['We'][0], cp['We'][1]])


def kernel(x_vehicle, x_rsu, edge_index_v2v, v2i_src, v2i_dst, i2v_src, i2v_dst,
           edge_attr_v2v, edge_attr_v2i, edge_attr_i2v, params):
    p = params
    xv = jnp.pad(x_vehicle, ((0, _NVP - _NV), (0, 0)))
    xu = jnp.pad(x_rsu, ((0, _NRP - _NR), (0, 0)))
    sv, dv = edge_index_v2v[0], edge_index_v2v[1]

    edv, epv = _pad_edges(sv, dv, edge_attr_v2v, _NV)
    edi, epi = _pad_edges(i2v_src, i2v_dst, edge_attr_i2v, _NV)
    edb, epb = _pad_edges(v2i_src, v2i_dst, edge_attr_v2i, _NR)

    s0 = _total_sum(edge_attr_v2v[:, 0].reshape(2500, 128))
    s1 = _total_sum(edge_attr_v2v[:, 1].reshape(2500, 128))
    s = jnp.concatenate([s0, s1], axis=1)

    w1v = jnp.concatenate([p['c1_v2v']['Wl'], p['c1_v2v']['Wr'],
                           p['c1_i2v']['Wr'], p['c1_v2i']['Wl']], axis=1)
    b1v = jnp.concatenate([p['c1_v2v']['bl'], p['c1_v2v']['br'],
                           p['c1_i2v']['br'], p['c1_v2i']['bl']])
    xl_v2v, xr_v2v, xr_i2v, xl_v2i = _mm(xv, w1v, b1v, 2048)
    w1r = jnp.concatenate([p['c1_i2v']['Wl'], p['c1_v2i']['Wr']], axis=1)
    b1r = jnp.concatenate([p['c1_i2v']['bl'], p['c1_v2i']['br']])
    xl_i2v, xr_v2i = _mm(xu, w1r, b1r, 512)

    def _cat(num, den):
        n = jnp.concatenate([num[0, :_NHALF], num[1, :_NHALF]], axis=0)
        d2 = den.reshape(2, _ACC_VL, 2)
        d = jnp.concatenate([d2[0, :_NHALF], d2[1, :_NHALF]], axis=0)
        return n, d

    nv1, dv1 = _cat(*_edge_pass(2, _ACC_VL, epv, True, _NHALF)(
        xl_v2v, xr_v2v, edv, _wmat(p['c1_v2v'])))
    ni1, di1 = _cat(*_edge_pass(2, _ACC_VL, epi, True, _NHALF)(
        xl_i2v, xr_i2v, edi, _wmat(p['c1_i2v'])))
    nb1, db1 = _edge_pass(2, _ACC_R, epb, False)(
        xl_v2i, xr_v2i, edb, _wmat(p['c1_v2i']))

    v1 = _combine1_veh(nv1, dv1, ni1, di1, xl_v2v, xr_v2v,
                       _wmat(p['c1_v2v']), s,
                       p['c1_v2v']['bias'].reshape(1, -1),
                       p['c1_i2v']['bias'].reshape(1, -1))
    r1 = _combine1_rsu(nb1, db1.reshape(2, _ACC_R, 2),
                       p['c1_v2i']['bias'].reshape(1, -1))

    w2v = jnp.concatenate([p['c2_v2v']['Wl'], p['c2_v2v']['Wr'],
                           p['c2_i2v']['Wr'], p['c2_v2i']['Wl']], axis=1)
    b2v = jnp.concatenate([p['c2_v2v']['bl'], p['c2_v2v']['br'],
                           p['c2_i2v']['br'], p['c2_v2i']['bl']])
    xl2_v2v, xr2_v2v, xr2_i2v, xl2_v2i = _mm(v1, w2v, b2v, 2048)
    w2r = jnp.concatenate([p['c2_i2v']['Wl'], p['c2_v2i']['Wr']], axis=1)
    b2r = jnp.concatenate([p['c2_i2v']['bl'], p['c2_v2i']['br']])
    xl2_i2v, xr2_v2i = _mm(r1, w2r, b2r, 512)

    nv2, dv2 = _cat(*_edge_pass(1, _ACC_VL, epv, True, _NHALF)(
        xl2_v2v, xr2_v2v, edv, _wmat(p['c2_v2v'])))
    ni2, di2 = _cat(*_edge_pass(1, _ACC_VL, epi, True, _NHALF)(
        xl2_i2v, xr2_i2v, edi, _wmat(p['c2_i2v'])))
    nb2, db2 = _edge_pass(1, _ACC_R, epb, False)(
        xl2_v2i, xr2_v2i, edb, _wmat(p['c2_v2i']))

    v2 = _combine2_veh(nv2, dv2, ni2, di2, xl2_v2v, xr2_v2v,
                       _wmat(p['c2_v2v']), s,
                       p['c2_v2v']['bias'].reshape(1, -1),
                       p['c2_i2v']['bias'].reshape(1, -1),
                       p['ln_veh_g'].reshape(1, -1), p['ln_veh_b'].reshape(1, -1))
    r2 = _combine2_rsu(nb2, db2.reshape(2, _ACC_R, 2),
                       p['c2_v2i']['bias'].reshape(1, -1),
                       p['ln_rsu_g'].reshape(1, -1), p['ln_rsu_b'].reshape(1, -1))
    return (v2[:_NV], r2[:_NR])


# speedup vs baseline: 1.8435x; 1.4853x over previous
import dataclasses
import functools

import jax
import jax.numpy as jnp
from jax import lax
from jax.experimental import pallas as pl
from jax.experimental.pallas import tpu as pltpu
from jax.experimental.pallas import tpu_sc as plsc

_NV, _NR = 10000, 500
_EV = 320000
_NVP = 10240
_NRP = 512
_NHALF = 5120
_ACC_VL = 6144
_ACC_R = 2048
_C = 96
_P = 208
_ZR = 64
_EPS = 1e-16

_SC_PARAMS = pltpu.CompilerParams()
if "needs_layout_passes" in pltpu.CompilerParams.__dataclass_fields__:
    _SC_PARAMS = dataclasses.replace(_SC_PARAMS, needs_layout_passes=False)


def _edge_pass(heads, n_dst_pad, e_pad, split_dst, n_half=0):
    nworkers = 16 if split_dst else 32
    ept = e_pad // nworkers
    nchunks = ept // _C
    rows_sub = n_dst_pad // 16
    ncopies = rows_sub // _ZR
    nd2r = (n_dst_pad * 2) // 128
    mesh = plsc.VectorSubcoreMesh(core_axis_name="c", subcore_axis_name="s",
                                  num_cores=2, num_subcores=16)

    def body(xl_hbm, xr_hbm, ed_hbm, w_hbm, num_hbm, den_hbm,
             ed0, ed1, src0, src1, dstg0, dstg1, dst20, dst21,
             ea00, ea01, ea10, ea11, w_v, xl0, xl1, xr0, xr1, msg_v, den_t,
             idx_r, acc_sh, accd_sh,
             semi0, semi1, semg0, semg1):
        cid = lax.axis_index("c")
        sid = lax.axis_index("s")
        wid = sid if split_dst else cid * 16 + sid
        lane = lax.iota(jnp.int32, 16)
        lane01 = jnp.minimum(lane, 1)
        zeros = jnp.zeros((16,), jnp.float32)
        ed = [ed0, ed1]
        src = [src0, src1]
        dstg = [dstg0, dstg1]
        dst2 = [dst20, dst21]
        ea0 = [ea00, ea01]
        ea1 = [ea10, ea11]
        xl = [xl0, xl1]
        xr = [xr0, xr1]
        semi = [semi0, semi1]
        semg = [semg0, semg1]

        @pl.loop(0, _C)
        def _z(i):
            for j in range(8):
                msg_v[i, pl.ds(j * 16, 16)] = zeros

        @pl.loop(0, ncopies)
        def _za(k):
            r0 = sid * rows_sub + k * _ZR
            pltpu.sync_copy(msg_v.at[pl.ds(0, _ZR)], acc_sh.at[pl.ds(r0, _ZR)])

        @pl.when(sid == 0)
        def _zd():
            off = 0
            while off < nd2r:
                sz = min(_ZR, nd2r - off)
                pltpu.sync_copy(msg_v.at[pl.ds(0, sz)], accd_sh.at[pl.ds(off, sz)])
                off += sz

        @pl.loop(0, nd2r)
        def _zt(i):
            for j in range(8):
                den_t[i, pl.ds(j * 16, 16)] = zeros

        @pl.loop(0, nd2r // 16)
        def _zi(k):
            idx_r[pl.ds(k * 16, 16)] = lane + k * 16

        pltpu.sync_copy(w_hbm, w_v)
        plsc.subcore_barrier()

        att = [w_v[0, pl.ds(j * 16, 16)] for j in range(8)]
        we0 = [w_v[1, pl.ds(j * 16, 16)] for j in range(8)]
        we1 = [w_v[2, pl.ds(j * 16, 16)] for j in range(8)]

        def fetch_ed(i, b):
            blk = jnp.minimum(wid * nchunks + i, (wid + 1) * nchunks - 1)
            pltpu.async_copy(ed_hbm.at[blk], ed[b], semi[b])

        def wait_ed(b):
            pltpu.make_async_copy(ed_hbm.at[0], ed[b], semi[b]).wait()

        def compute(ea0_r, ea1_r, dst_r, xl_r, xr_r):
            @pl.loop(0, _C)
            def _edge(e):
                ei = jnp.full((16,), e, jnp.int32)
                a0 = plsc.bitcast(plsc.load_gather(ea0_r, [ei]), jnp.float32)
                a1 = plsc.bitcast(plsc.load_gather(ea1_r, [ei]), jnp.float32)
                dstv = plsc.load_gather(dst_r, [ei])
                xls = []
                s0 = zeros
                s1 = zeros
                for j in range(8):
                    xlj = xl_r[e, pl.ds(j * 16, 16)]
                    xrj = xr_r[e, pl.ds(j * 16, 16)]
                    m = xlj + xrj + a0 * we0[j] + a1 * we1[j]
                    m = jnp.maximum(m, 0.2 * m)
                    if heads == 2 and j >= 4:
                        s1 = s1 + m * att[j]
                    else:
                        s0 = s0 + m * att[j]
                    xls.append(xlj)
                if heads == 2:
                    e0 = jnp.exp(jnp.full((16,), jnp.sum(s0), jnp.float32))
                    e1 = jnp.exp(jnp.full((16,), jnp.sum(s1), jnp.float32))
                    for j in range(8):
                        msg_v[e, pl.ds(j * 16, 16)] = xls[j] * (e0 if j < 4 else e1)
                    den = jnp.where(lane == 0, e0, jnp.where(lane == 1, e1, zeros))
                else:
                    e0 = jnp.exp(jnp.full((16,), jnp.sum(s0), jnp.float32))
                    for j in range(8):
                        msg_v[e, pl.ds(j * 16, 16)] = xls[j] * e0
                    den = jnp.where(lane == 0, e0, zeros)
                flat = dstv * 2 + lane01
                plsc.addupdate_scatter(
                    den_t,
                    [lax.shift_right_logical(flat, 7),
                     lax.bitwise_and(flat, 127)],
                    den, mask=lane < heads)

        if split_dst:
            pend = [src0, dstg0, ea00, ea10]

            def process():
                @pl.loop(0, _C // 16)
                def _cp(g):
                    sl = pl.ds(g * 16, 16)
                    d = dstg0[sl]
                    dst20[sl] = d
                    dst21[sl] = jnp.minimum(d + cid * n_half, 2 * n_half - 1)
                d1 = pltpu.async_copy(xl_hbm.at[src0.at[pl.ds(0, _C)]],
                                      xl0, semg0)
                d2 = pltpu.async_copy(xr_hbm.at[dst21.at[pl.ds(0, _C)]],
                                      xr0, semg1)
                d1.wait()
                d2.wait()
                compute(ea00, ea10, dst20, xl0, xr0)
                pltpu.sync_copy(msg_v, acc_sh.at[dst20], add=True)
                for arr in pend:
                    for g in range(7):
                        sl = pl.ds(g * 16, 16)
                        arr[sl] = arr[pl.ds(_C + g * 16, 16)]

            fetch_ed(0, 0)
            fetch_ed(1, 1)

            def _pair(tt, fill):
                for b in (0, 1):
                    i = tt * 2 + b
                    wait_ed(b)
                    for g in range(_C // 16):
                        sl = pl.ds(g * 16, 16)
                        d_ = ed[b][1, sl]
                        dl = d_ - cid * n_half
                        ok = (dl >= 0) & (dl < n_half)
                        at = pl.ds(fill, 16)
                        plsc.store_compressed(src0.at[at], ed[b][0, sl], mask=ok)
                        plsc.store_compressed(dstg0.at[at], dl, mask=ok)
                        plsc.store_compressed(ea00.at[at], ed[b][2, sl], mask=ok)
                        plsc.store_compressed(ea10.at[at], ed[b][3, sl], mask=ok)
                        cnt = plsc.all_reduce_population_count(ok)
                        fill = fill + jnp.max(cnt)
                    fetch_ed(i + 2, b)

                    @pl.when(fill >= _C)
                    def _go():
                        process()

                    fill = fill - jnp.where(fill >= _C, _C, 0)
                return fill

            fill = lax.fori_loop(0, nchunks // 2, _pair, 0)

            fillv = jnp.full((16,), fill, jnp.int32)

            @pl.loop(0, _C // 16)
            def _pad(g):
                sl = pl.ds(g * 16, 16)
                gl = lane + g * 16
                padm = gl >= fillv
                src0[sl] = jnp.where(padm, 0, src0[sl])
                dstg0[sl] = jnp.where(padm, n_half, dstg0[sl])
                ea00[sl] = jnp.where(padm, 0, ea00[sl])
                ea10[sl] = jnp.where(padm, 0, ea10[sl])

            process()
            wait_ed(0)
            wait_ed(1)
        else:
            def prep(b):
                @pl.loop(0, _C // 16)
                def _p(g):
                    sl = pl.ds(g * 16, 16)
                    src[b][sl] = ed[b][0, sl]
                    d = ed[b][1, sl]
                    dstg[b][sl] = d
                    ea0[b][sl] = ed[b][2, sl]
                    ea1[b][sl] = ed[b][3, sl]
                    dst2[b][sl] = d

            def fire_gathers(b):
                d1 = pltpu.async_copy(xl_hbm.at[src[b]], xl[b], semg[b])
                d2 = pltpu.async_copy(xr_hbm.at[dstg[b]], xr[b], semg[b])
                return d1, d2

            fetch_ed(0, 0)
            fetch_ed(1, 1)

            @pl.loop(0, nchunks // 2)
            def _pairs(tt):
                i = tt * 2
                wait_ed(0)
                prep(0)
                da = fire_gathers(0)
                wait_ed(1)
                prep(1)
                db = fire_gathers(1)
                da[0].wait()
                da[1].wait()
                compute(ea0[0], ea1[0], dst2[0], xl0, xr0)
                pltpu.sync_copy(msg_v, acc_sh.at[dst2[0]], add=True)
                fetch_ed(i + 2, 0)
                db[0].wait()
                db[1].wait()
                compute(ea0[1], ea1[1], dst2[1], xl1, xr1)
                pltpu.sync_copy(msg_v, acc_sh.at[dst2[1]], add=True)
                fetch_ed(i + 3, 1)

            if nchunks % 2:
                wait_ed(0)
                prep(0)
                da = fire_gathers(0)
                da[0].wait()
                da[1].wait()
                compute(ea0[0], ea1[0], dst2[0], xl0, xr0)
                pltpu.sync_copy(msg_v, acc_sh.at[dst2[0]], add=True)
                wait_ed(1)
            else:
                wait_ed(0)
                wait_ed(1)

        pltpu.sync_copy(den_t, accd_sh.at[idx_r], add=True)
        plsc.subcore_barrier()

        @pl.loop(0, ncopies)
        def _wb(k):
            r0 = sid * rows_sub + k * _ZR
            pltpu.sync_copy(acc_sh.at[pl.ds(r0, _ZR)], num_hbm.at[cid, pl.ds(r0, _ZR)])

        @pl.when(sid == 0)
        def _wbd():
            pltpu.sync_copy(accd_sh, den_hbm.at[cid])

    return pl.kernel(
        body,
        out_type=(jax.ShapeDtypeStruct((2, n_dst_pad, 128), jnp.float32),
                  jax.ShapeDtypeStruct((2, nd2r, 128), jnp.float32)),
        mesh=mesh,
        compiler_params=_SC_PARAMS,
        scratch_types=(
            [pltpu.VMEM((4, _C), jnp.int32)] * 2
            + ([pltpu.VMEM((_P,), jnp.int32),
                pltpu.VMEM((16,), jnp.int32),
                pltpu.VMEM((_P,), jnp.int32),
                pltpu.VMEM((16,), jnp.int32),
                pltpu.VMEM((_C,), jnp.int32),
                pltpu.VMEM((_P,), jnp.int32),
                pltpu.VMEM((_P,), jnp.int32),
                pltpu.VMEM((16,), jnp.int32),
                pltpu.VMEM((_P,), jnp.int32),
                pltpu.VMEM((16,), jnp.int32)]
               if split_dst else
               [pltpu.VMEM((_C,), jnp.int32)] * 10)
            + [pltpu.VMEM((3, 128), jnp.float32)]
            + ([pltpu.VMEM((_C, 128), jnp.float32),
                pltpu.VMEM((8, 128), jnp.float32),
                pltpu.VMEM((_C, 128), jnp.float32),
                pltpu.VMEM((8, 128), jnp.float32),
                pltpu.VMEM((_C, 128), jnp.float32)]
               if split_dst else
               [pltpu.VMEM((_C, 128), jnp.float32)] * 5)
            + [pltpu.VMEM((nd2r, 128), jnp.float32),
               pltpu.VMEM((nd2r,), jnp.int32),
               pltpu.VMEM_SHARED((n_dst_pad, 128), jnp.float32),
               pltpu.VMEM_SHARED((nd2r, 128), jnp.float32)]
            + [pltpu.SemaphoreType.DMA] * 4
        ),
    )


def _mm(x, w, b, bm):
    m, k = x.shape
    n = w.shape[1]
    nout = n // 128

    def body(x_ref, w_ref, b_ref, *outs):
        y = jnp.dot(x_ref[...], w_ref[...], preferred_element_type=jnp.float32)
        y = y + b_ref[...]
        for i, o in enumerate(outs):
            o[...] = y[:, i * 128:(i + 1) * 128]

    return pl.pallas_call(
        body,
        grid=(m // bm,),
        in_specs=[
            pl.BlockSpec((bm, k), lambda i: (i, 0)),
            pl.BlockSpec((k, n), lambda i: (0, 0)),
            pl.BlockSpec((1, n), lambda i: (0, 0)),
        ],
        out_specs=[pl.BlockSpec((bm, 128), lambda i: (i, 0)) for _ in range(nout)],
        out_shape=[jax.ShapeDtypeStruct((m, 128), jnp.float32) for _ in range(nout)],
    )(x, w, b.reshape(1, n))


def _total_sum(x2d):
    def body(x_ref, o_ref):
        o_ref[...] = jnp.sum(x_ref[...])[None, None]

    return pl.pallas_call(
        body, out_shape=jax.ShapeDtypeStruct((1, 1), jnp.float32))(x2d)


def _combine1_veh(numv, denv, numi, deni, xl, xr, w, s, bias_v, bias_i):
    bm = 1024

    def body(nv_ref, dv_ref, ni_ref, di_ref, xl_ref, xr_ref, w_ref, s_ref,
             bv_ref, bi_ref, o_ref):
        xl_ = xl_ref[...]
        w_ = w_ref[...]
        xe = (s_ref[0, 0] / _EV) * w_[1:2, :] + (s_ref[0, 1] / _EV) * w_[2:3, :]
        m = xl_ + xr_ref[...] + xe
        m = jnp.maximum(m, 0.2 * m)
        am = m * w_[0:1, :]
        e0 = jnp.exp(jnp.sum(am[:, :64], axis=1, keepdims=True))
        e1 = jnp.exp(jnp.sum(am[:, 64:], axis=1, keepdims=True))
        av = nv_ref[...]
        dv = dv_ref[...]
        num = av + jnp.concatenate([e0 * xl_[:, :64], e1 * xl_[:, 64:]], axis=1)
        ov = jnp.concatenate(
            [num[:, :64] / (dv[:, 0:1] + e0 + _EPS),
             num[:, 64:] / (dv[:, 1:2] + e1 + _EPS)], axis=1) + bv_ref[...]
        ai = ni_ref[...]
        di = di_ref[...]
        oi = jnp.concatenate(
            [ai[:, :64] / (di[:, 0:1] + _EPS),
             ai[:, 64:] / (di[:, 1:2] + _EPS)], axis=1) + bi_ref[...]
        v = ov + oi
        o_ref[...] = jnp.where(v > 0, v, jnp.exp(v) - 1.0)

    return pl.pallas_call(
        body,
        grid=(_NVP // bm,),
        in_specs=[
            pl.BlockSpec((bm, 128), lambda i: (i, 0)),
            pl.BlockSpec((bm, 2), lambda i: (i, 0)),
            pl.BlockSpec((bm, 128), lambda i: (i, 0)),
            pl.BlockSpec((bm, 2), lambda i: (i, 0)),
            pl.BlockSpec((bm, 128), lambda i: (i, 0)),
            pl.BlockSpec((bm, 128), lambda i: (i, 0)),
            pl.BlockSpec((3, 128), lambda i: (0, 0)),
            pl.BlockSpec(memory_space=pltpu.SMEM),
            pl.BlockSpec((1, 128), lambda i: (0, 0)),
            pl.BlockSpec((1, 128), lambda i: (0, 0)),
        ],
        out_specs=pl.BlockSpec((bm, 128), lambda i: (i, 0)),
        out_shape=jax.ShapeDtypeStruct((_NVP, 128), jnp.float32),
    )(numv, denv, numi, deni, xl, xr, w, s, bias_v, bias_i)


def _combine1_rsu(num, den, bias):
    def body(n_ref, d_ref, b_ref, o_ref):
        a = n_ref[0] + n_ref[1]
        d = d_ref[0] + d_ref[1]
        o = jnp.concatenate(
            [a[:, :64] / (d[:, 0:1] + _EPS),
             a[:, 64:] / (d[:, 1:2] + _EPS)], axis=1) + b_ref[...]
        o_ref[...] = jnp.where(o > 0, o, jnp.exp(o) - 1.0)

    return pl.pallas_call(
        body,
        grid=(1,),
        in_specs=[
            pl.BlockSpec((2, _NRP, 128), lambda i: (0, 0, 0)),
            pl.BlockSpec((2, _NRP, 2), lambda i: (0, 0, 0)),
            pl.BlockSpec((1, 128), lambda i: (0, 0)),
        ],
        out_specs=pl.BlockSpec((_NRP, 128), lambda i: (0, 0)),
        out_shape=jax.ShapeDtypeStruct((_NRP, 128), jnp.float32),
    )(num, den, bias)


def _combine2_veh(numv, denv, numi, deni, xl, xr, w, s, bias_v, bias_i, g, b):
    bm = 1024

    def body(nv_ref, dv_ref, ni_ref, di_ref, xl_ref, xr_ref, w_ref, s_ref,
             bv_ref, bi_ref, g_ref, be_ref, o_ref):
        xl_ = xl_ref[...]
        w_ = w_ref[...]
        xe = (s_ref[0, 0] / _EV) * w_[1:2, :] + (s_ref[0, 1] / _EV) * w_[2:3, :]
        m = xl_ + xr_ref[...] + xe
        m = jnp.maximum(m, 0.2 * m)
        e = jnp.exp(jnp.sum(m * w_[0:1, :], axis=1, keepdims=True))
        av = nv_ref[...]
        dv = dv_ref[...]
        ov = (av + e * xl_) / (dv[:, 0:1] + e + _EPS) + bv_ref[...]
        ai = ni_ref[...]
        di = di_ref[...]
        oi = ai / (di[:, 0:1] + _EPS) + bi_ref[...]
        v = ov + oi
        mu = jnp.mean(v, axis=1, keepdims=True)
        cv = v - mu
        var = jnp.mean(cv * cv, axis=1, keepdims=True)
        o_ref[...] = cv * lax.rsqrt(var + 1e-5) * g_ref[...] + be_ref[...]

    return pl.pallas_call(
        body,
        grid=(_NVP // bm,),
        in_specs=[
            pl.BlockSpec((bm, 128), lambda i: (i, 0)),
            pl.BlockSpec((bm, 2), lambda i: (i, 0)),
            pl.BlockSpec((bm, 128), lambda i: (i, 0)),
            pl.BlockSpec((bm, 2), lambda i: (i, 0)),
            pl.BlockSpec((bm, 128), lambda i: (i, 0)),
            pl.BlockSpec((bm, 128), lambda i: (i, 0)),
            pl.BlockSpec((3, 128), lambda i: (0, 0)),
            pl.BlockSpec(memory_space=pltpu.SMEM),
            pl.BlockSpec((1, 128), lambda i: (0, 0)),
            pl.BlockSpec((1, 128), lambda i: (0, 0)),
            pl.BlockSpec((1, 128), lambda i: (0, 0)),
            pl.BlockSpec((1, 128), lambda i: (0, 0)),
        ],
        out_specs=pl.BlockSpec((bm, 128), lambda i: (i, 0)),
        out_shape=jax.ShapeDtypeStruct((_NVP, 128), jnp.float32),
    )(numv, denv, numi, deni, xl, xr, w, s, bias_v, bias_i, g, b)


def _combine2_rsu(num, den, bias, g, b):
    def body(n_ref, d_ref, b_ref, g_ref, be_ref, o_ref):
        a = n_ref[0] + n_ref[1]
        d = d_ref[0] + d_ref[1]
        v = a / (d[:, 0:1] + _EPS) + b_ref[...]
        mu = jnp.mean(v, axis=1, keepdims=True)
        cv = v - mu
        var = jnp.mean(cv * cv, axis=1, keepdims=True)
        o_ref[...] = cv * lax.rsqrt(var + 1e-5) * g_ref[...] + be_ref[...]

    return pl.pallas_call(
        body,
        grid=(1,),
        in_specs=[
            pl.BlockSpec((2, _NRP, 128), lambda i: (0, 0, 0)),
            pl.BlockSpec((2, _NRP, 2), lambda i: (0, 0, 0)),
            pl.BlockSpec((1, 128), lambda i: (0, 0)),
            pl.BlockSpec((1, 128), lambda i: (0, 0)),
            pl.BlockSpec((1, 128), lambda i: (0, 0)),
        ],
        out_specs=pl.BlockSpec((_NRP, 128), lambda i: (0, 0)),
        out_shape=jax.ShapeDtypeStruct((_NRP, 128), jnp.float32),
    )(num, den, bias, g, b)


def _pad_edges(src, dst, ea, n_dst):
    e = src.shape[0]
    ep = -(-e // 3072) * 3072
    pad = ep - e
    src_p = jnp.concatenate([src.astype(jnp.int32), jnp.zeros((pad,), jnp.int32)])
    dst_p = jnp.concatenate([dst.astype(jnp.int32),
                             jnp.full((pad,), n_dst, jnp.int32)])
    ea0 = jnp.concatenate([ea[:, 0], jnp.zeros((pad,), jnp.float32)])
    ea1 = jnp.concatenate([ea[:, 1], jnp.zeros((pad,), jnp.float32)])
    edata = jnp.stack([src_p.reshape(-1, _C),
                       dst_p.reshape(-1, _C),
                       lax.bitcast_convert_type(ea0, jnp.int32).reshape(-1, _C),
                       lax.bitcast_convert_type(ea1, jnp.int32).reshape(-1, _C)],
                      axis=1)
    return edata, ep


def _wmat(cp):
    return jnp.stack([cp['att'].reshape(-1), cp['We'][0], cp['We'][1]])


def kernel(x_vehicle, x_rsu, edge_index_v2v, v2i_src, v2i_dst, i2v_src, i2v_dst,
           edge_attr_v2v, edge_attr_v2i, edge_attr_i2v, params):
    p = params
    xv = jnp.pad(x_vehicle, ((0, _NVP - _NV), (0, 0)))
    xu = jnp.pad(x_rsu, ((0, _NRP - _NR), (0, 0)))
    sv, dv = edge_index_v2v[0], edge_index_v2v[1]

    edv, epv = _pad_edges(sv, dv, edge_attr_v2v, _NV)
    edi, epi = _pad_edges(i2v_src, i2v_dst, edge_attr_i2v, _NV)
    edb, epb = _pad_edges(v2i_src, v2i_dst, edge_attr_v2i, _NR)

    s0 = _total_sum(edge_attr_v2v[:, 0].reshape(2500, 128))
    s1 = _total_sum(edge_attr_v2v[:, 1].reshape(2500, 128))
    s = jnp.concatenate([s0, s1], axis=1)

    w1v = jnp.concatenate([p['c1_v2v']['Wl'], p['c1_v2v']['Wr'],
                           p['c1_i2v']['Wr'], p['c1_v2i']['Wl']], axis=1)
    b1v = jnp.concatenate([p['c1_v2v']['bl'], p['c1_v2v']['br'],
                           p['c1_i2v']['br'], p['c1_v2i']['bl']])
    xl_v2v, xr_v2v, xr_i2v, xl_v2i = _mm(xv, w1v, b1v, 2048)
    w1r = jnp.concatenate([p['c1_i2v']['Wl'], p['c1_v2i']['Wr']], axis=1)
    b1r = jnp.concatenate([p['c1_i2v']['bl'], p['c1_v2i']['br']])
    xl_i2v, xr_v2i = _mm(xu, w1r, b1r, 512)

    def _cat(num, den):
        n = jnp.concatenate([num[0, :_NHALF], num[1, :_NHALF]], axis=0)
        d2 = den.reshape(2, _ACC_VL, 2)
        d = jnp.concatenate([d2[0, :_NHALF], d2[1, :_NHALF]], axis=0)
        return n, d

    nv1, dv1 = _cat(*_edge_pass(2, _ACC_VL, epv, True, _NHALF)(
        xl_v2v, xr_v2v, edv, _wmat(p['c1_v2v'])))
    ni1, di1 = _cat(*_edge_pass(2, _ACC_VL, epi, True, _NHALF)(
        xl_i2v, xr_i2v, edi, _wmat(p['c1_i2v'])))
    nb1, db1 = _edge_pass(2, _ACC_R, epb, False)(
        xl_v2i, xr_v2i, edb, _wmat(p['c1_v2i']))

    v1 = _combine1_veh(nv1, dv1, ni1, di1, xl_v2v, xr_v2v,
                       _wmat(p['c1_v2v']), s,
                       p['c1_v2v']['bias'].reshape(1, -1),
                       p['c1_i2v']['bias'].reshape(1, -1))
    r1 = _combine1_rsu(nb1, db1.reshape(2, _ACC_R, 2),
                       p['c1_v2i']['bias'].reshape(1, -1))

    w2v = jnp.concatenate([p['c2_v2v']['Wl'], p['c2_v2v']['Wr'],
                           p['c2_i2v']['Wr'], p['c2_v2i']['Wl']], axis=1)
    b2v = jnp.concatenate([p['c2_v2v']['bl'], p['c2_v2v']['br'],
                           p['c2_i2v']['br'], p['c2_v2i']['bl']])
    xl2_v2v, xr2_v2v, xr2_i2v, xl2_v2i = _mm(v1, w2v, b2v, 2048)
    w2r = jnp.concatenate([p['c2_i2v']['Wl'], p['c2_v2i']['Wr']], axis=1)
    b2r = jnp.concatenate([p['c2_i2v']['bl'], p['c2_v2i']['br']])
    xl2_i2v, xr2_v2i = _mm(r1, w2r, b2r, 512)

    nv2, dv2 = _cat(*_edge_pass(1, _ACC_VL, epv, True, _NHALF)(
        xl2_v2v, xr2_v2v, edv, _wmat(p['c2_v2v'])))
    ni2, di2 = _cat(*_edge_pass(1, _ACC_VL, epi, True, _NHALF)(
        xl2_i2v, xr2_i2v, edi, _wmat(p['c2_i2v'])))
    nb2, db2 = _edge_pass(1, _ACC_R, epb, False)(
        xl2_v2i, xr2_v2i, edb, _wmat(p['c2_v2i']))

    v2 = _combine2_veh(nv2, dv2, ni2, di2, xl2_v2v, xr2_v2v,
                       _wmat(p['c2_v2v']), s,
                       p['c2_v2v']['bias'].reshape(1, -1),
                       p['c2_i2v']['bias'].reshape(1, -1),
                       p['ln_veh_g'].reshape(1, -1), p['ln_veh_b'].reshape(1, -1))
    r2 = _combine2_rsu(nb2, db2.reshape(2, _ACC_R, 2),
                       p['c2_v2i']['bias'].reshape(1, -1),
                       p['ln_rsu_g'].reshape(1, -1), p['ln_rsu_b'].reshape(1, -1))
    return (v2[:_NV], r2[:_NR])
